# baseline (speedup 1.0000x reference)
import functools

import jax
import jax.numpy as jnp
from jax import lax
from jax.experimental import pallas as pl
from jax.experimental.pallas import tpu as pltpu
from jax.experimental.pallas import tpu_sc as plsc

N = 10000
E = 320000
D = 128
EPS = 1e-5
NEG_SLOPE = 0.01

NC, NS = 2, 16
NW = NC * NS
CH = 128
NCH = 80
EPT = NCH * CH
EP = NW * EPT
ECH = EP // CH

BM = 400
GRID = N // BM
NP = 10240

_mesh = plsc.VectorSubcoreMesh(
    core_axis_name="c", subcore_axis_name="s", num_cores=NC, num_subcores=NS)


@functools.partial(
    pl.kernel,
    out_type=jax.ShapeDtypeStruct((NC * NP,), jnp.float32),
    mesh=_mesh,
    scratch_types=[
        pltpu.VMEM_SHARED((NP,), jnp.float32),
        pltpu.VMEM((NCH, CH), jnp.int32),
        pltpu.VMEM((CH,), jnp.float32),
        pltpu.VMEM((NP // NS,), jnp.float32),
    ],
)
def _deg_kernel(dst2d_hbm, out_hbm, acc, didx, ones, zbuf):
    c = lax.axis_index("c")
    s = lax.axis_index("s")
    wid = s * NC + c

    one16 = jnp.full((16,), 1.0, dtype=jnp.float32)
    zero16 = jnp.zeros((16,), dtype=jnp.float32)

    @pl.loop(0, CH // 16)
    def _(i):
        ones[pl.ds(i * 16, 16)] = one16

    @pl.loop(0, NP // NS // 16)
    def _(i):
        zbuf[pl.ds(i * 16, 16)] = zero16
    pltpu.sync_copy(zbuf, acc.at[pl.ds(s * (NP // NS), NP // NS)])

    pltpu.sync_copy(dst2d_hbm.at[pl.ds(wid * NCH, NCH)], didx)

    plsc.subcore_barrier()

    @pl.loop(0, NCH)
    def _(k):
        pltpu.sync_copy(ones, acc.at[didx.at[k]], add=True)

    plsc.subcore_barrier()

    pltpu.sync_copy(acc.at[pl.ds(s * (NP // NS), NP // NS)], zbuf)
    pltpu.sync_copy(zbuf, out_hbm.at[pl.ds(c * NP + s * (NP // NS), NP // NS)])


def _hs_body(deg_ref, x_ref, w_ref, hs_ref, dinv_ref):
    deg = 1.0 + deg_ref[0] + deg_ref[1]
    dinv = lax.rsqrt(deg)
    h = jnp.dot(x_ref[...], w_ref[...], preferred_element_type=jnp.float32)
    hs_ref[...] = h * dinv
    dinv_ref[...] = dinv


_hs_call = pl.pallas_call(
    _hs_body,
    grid=(GRID,),
    in_specs=[
        pl.BlockSpec((NC, BM, 1), lambda i: (0, i, 0)),
        pl.BlockSpec((BM, D), lambda i: (i, 0)),
        pl.BlockSpec((D, D), lambda i: (0, 0)),
    ],
    out_specs=[
        pl.BlockSpec((BM, D), lambda i: (i, 0)),
        pl.BlockSpec((BM, 1), lambda i: (i, 0)),
    ],
    out_shape=[
        jax.ShapeDtypeStruct((N, D), jnp.float32),
        jax.ShapeDtypeStruct((N, 1), jnp.float32),
    ],
)


@functools.partial(
    pl.kernel,
    out_type=jax.ShapeDtypeStruct((NC * NP, D), jnp.float32),
    mesh=_mesh,
    scratch_types=[
        pltpu.VMEM_SHARED((NP, D), jnp.float32),
        pltpu.VMEM((CH,), jnp.int32),
        pltpu.VMEM((CH,), jnp.int32),
        pltpu.VMEM((CH,), jnp.int32),
        pltpu.VMEM((CH,), jnp.int32),
        pltpu.VMEM((CH, D), jnp.float32),
        pltpu.VMEM((CH, D), jnp.float32),
        pltpu.SemaphoreType.DMA,
        pltpu.SemaphoreType.DMA,
        pltpu.SemaphoreType.DMA,
        pltpu.SemaphoreType.DMA,
    ],
)
def _agg_kernel(src_hbm, dst_hbm, hs_hbm, out_hbm, acc, sidx0, sidx1,
                didx0, didx1, rows0, rows1, isem0, isem1, gsem0, gsem1):
    c = lax.axis_index("c")
    s = lax.axis_index("s")
    wid = s * NC + c
    sbufs = (sidx0, sidx1)
    dbufs = (didx0, didx1)
    rbufs = (rows0, rows1)
    isems = (isem0, isem1)
    gsems = (gsem0, gsem1)

    zero16 = jnp.zeros((16,), dtype=jnp.float32)

    @pl.loop(0, CH)
    def _(r):
        @pl.loop(0, D // 16)
        def _(j):
            rows0[r, pl.ds(j * 16, 16)] = zero16

    rbase = s * (NP // NS)
    for k in range(5):
        pltpu.sync_copy(rows0, acc.at[pl.ds(rbase + k * CH, CH)])

    plsc.subcore_barrier()

    ebase = wid * EPT

    def _idx_start(chunk, b):
        pltpu.async_copy(src_hbm.at[pl.ds(ebase + chunk * CH, CH)],
                         sbufs[b], isems[b])
        pltpu.async_copy(dst_hbm.at[pl.ds(ebase + chunk * CH, CH)],
                         dbufs[b], isems[b])

    def _idx_wait(chunk, b):
        pltpu.make_async_copy(src_hbm.at[pl.ds(ebase + chunk * CH, CH)],
                              sbufs[b], isems[b]).wait()
        pltpu.make_async_copy(dst_hbm.at[pl.ds(ebase + chunk * CH, CH)],
                              dbufs[b], isems[b]).wait()

    def _gather_start(b):
        pltpu.async_copy(hs_hbm.at[sbufs[b]], rbufs[b], gsems[b])

    def _gather_wait(b):
        pltpu.make_async_copy(hs_hbm.at[sbufs[b]], rbufs[b], gsems[b]).wait()

    _idx_start(0, 0)
    _idx_start(1, 1)
    _idx_wait(0, 0)
    _gather_start(0)

    @pl.loop(0, NCH, step=2)
    def _(k):
        for b in range(2):
            cur = k + b
            nb = 1 - b

            @pl.when(cur + 1 < NCH)
            def _():
                _idx_wait(cur + 1, nb)
                _gather_start(nb)

            _gather_wait(b)
            pltpu.sync_copy(rbufs[b], acc.at[dbufs[b]], add=True)

            @pl.when(cur + 2 < NCH)
            def _():
                _idx_start(cur + 2, b)

    plsc.subcore_barrier()

    for k in range(5):
        pltpu.sync_copy(acc.at[pl.ds(rbase + k * CH, CH)], rows0)
        pltpu.sync_copy(rows0, out_hbm.at[pl.ds(c * NP + rbase + k * CH, CH)])


def _fin_body(y_ref, hs_ref, dinv_ref, b_ref, gamma_ref, beta_ref, out_ref,
              acc_s, acc_q, scale_s, shift_s):
    p = pl.program_id(0)
    i = pl.program_id(1)
    t = (y_ref[0] + y_ref[1] + hs_ref[...]) * dinv_ref[...] + b_ref[...]
    z = jnp.where(t >= 0, t, NEG_SLOPE * t)

    @pl.when(jnp.logical_and(p == 0, i == 0))
    def _():
        acc_s[...] = jnp.zeros_like(acc_s)
        acc_q[...] = jnp.zeros_like(acc_q)

    @pl.when(p == 0)
    def _():
        acc_s[...] += jnp.sum(z, axis=0, keepdims=True)
        acc_q[...] += jnp.sum(z * z, axis=0, keepdims=True)
        out_ref[...] = z

    @pl.when(jnp.logical_and(p == 1, i == 0))
    def _():
        mean = acc_s[...] * (1.0 / N)
        var = acc_q[...] * (1.0 / N) - mean * mean
        g_rstd = gamma_ref[...] * lax.rsqrt(var + EPS)
        scale_s[...] = g_rstd
        shift_s[...] = beta_ref[...] - mean * g_rstd

    @pl.when(p == 1)
    def _():
        out_ref[...] = z * scale_s[...] + shift_s[...]


_fin_call = pl.pallas_call(
    _fin_body,
    grid=(2, GRID),
    in_specs=[
        pl.BlockSpec((NC, BM, D), lambda p, i: (0, i, 0)),
        pl.BlockSpec((BM, D), lambda p, i: (i, 0)),
        pl.BlockSpec((BM, 1), lambda p, i: (i, 0)),
        pl.BlockSpec((1, D), lambda p, i: (0, 0)),
        pl.BlockSpec((1, D), lambda p, i: (0, 0)),
        pl.BlockSpec((1, D), lambda p, i: (0, 0)),
    ],
    out_specs=pl.BlockSpec((BM, D), lambda p, i: (i, 0)),
    out_shape=jax.ShapeDtypeStruct((N, D), jnp.float32),
    scratch_shapes=[
        pltpu.VMEM((1, D), jnp.float32),
        pltpu.VMEM((1, D), jnp.float32),
        pltpu.VMEM((1, D), jnp.float32),
        pltpu.VMEM((1, D), jnp.float32),
    ],
)


def kernel(x, edge_index, W, b, gamma, beta):
    npad = EP - E
    src = jnp.concatenate(
        [edge_index[0], jnp.zeros((npad,), jnp.int32)])
    pad_dst = N + (jnp.arange(npad, dtype=jnp.int32) % (NP - N))
    dst = jnp.concatenate([edge_index[1], pad_dst])
    dst2d = dst.reshape(ECH, CH)

    degp = _deg_kernel(dst2d).reshape(NC, NP, 1)
    hs, dinv = _hs_call(degp, x, W)
    y = _agg_kernel(src, dst, hs).reshape(NC, NP, D)
    return _fin_call(y, hs, dinv, b.reshape(1, D), gamma.reshape(1, D),
                     beta.reshape(1, D))


# speedup vs baseline: 2.2569x; 2.2569x over previous
import functools

import jax
import jax.numpy as jnp
from jax import lax
from jax.experimental import pallas as pl
from jax.experimental.pallas import tpu as pltpu
from jax.experimental.pallas import tpu_sc as plsc

N = 10000
E = 320000
D = 128
EPS = 1e-5
NEG_SLOPE = 0.01

NC, NS = 2, 16
NW = NC * NS
CH = 128
ECH = E // CH
NCH = 80
ECH2D = NW * NCH

BM = 400
GRID = N // BM
NP = 10240

_mesh = plsc.VectorSubcoreMesh(
    core_axis_name="c", subcore_axis_name="s", num_cores=NC, num_subcores=NS)


@functools.partial(
    pl.kernel,
    out_type=jax.ShapeDtypeStruct((NC * NP,), jnp.float32),
    mesh=_mesh,
    scratch_types=[
        pltpu.VMEM_SHARED((NP,), jnp.float32),
        pltpu.VMEM((NCH, CH), jnp.int32),
        pltpu.VMEM((CH,), jnp.float32),
        pltpu.VMEM((NP // NS,), jnp.float32),
    ],
)
def _deg_kernel(dst2d_hbm, out_hbm, acc, didx, ones, zbuf):
    c = lax.axis_index("c")
    s = lax.axis_index("s")
    wid = s * NC + c

    one16 = jnp.full((16,), 1.0, dtype=jnp.float32)
    zero16 = jnp.zeros((16,), dtype=jnp.float32)

    @pl.loop(0, CH // 16)
    def _(i):
        ones[pl.ds(i * 16, 16)] = one16

    @pl.loop(0, NP // NS // 16)
    def _(i):
        zbuf[pl.ds(i * 16, 16)] = zero16
    pltpu.sync_copy(zbuf, acc.at[pl.ds(s * (NP // NS), NP // NS)])

    pltpu.sync_copy(dst2d_hbm.at[pl.ds(wid * NCH, NCH)], didx)

    plsc.subcore_barrier()

    @pl.loop(0, NCH)
    def _(k):
        pltpu.sync_copy(ones, acc.at[didx.at[k]], add=True)

    plsc.subcore_barrier()

    pltpu.sync_copy(acc.at[pl.ds(s * (NP // NS), NP // NS)], zbuf)
    pltpu.sync_copy(zbuf, out_hbm.at[pl.ds(c * NP + s * (NP // NS), NP // NS)])


def _hs_body(deg_ref, x_ref, w_ref, hs_ref, dinv_ref):
    deg = 1.0 + deg_ref[0] + deg_ref[1]
    dinv = lax.rsqrt(deg)
    h = jnp.dot(x_ref[...], w_ref[...], preferred_element_type=jnp.float32)
    hs_ref[...] = h * dinv
    dinv_ref[...] = dinv


_hs_call = pl.pallas_call(
    _hs_body,
    grid=(GRID,),
    in_specs=[
        pl.BlockSpec((NC, BM, 1), lambda i: (0, i, 0)),
        pl.BlockSpec((BM, D), lambda i: (i, 0)),
        pl.BlockSpec((D, D), lambda i: (0, 0)),
    ],
    out_specs=[
        pl.BlockSpec((BM, D), lambda i: (i, 0)),
        pl.BlockSpec((BM, 1), lambda i: (i, 0)),
    ],
    out_shape=[
        jax.ShapeDtypeStruct((N, D), jnp.float32),
        jax.ShapeDtypeStruct((N, 1), jnp.float32),
    ],
)


@functools.partial(
    pl.kernel,
    out_type=jax.ShapeDtypeStruct((NC * NP, D), jnp.float32),
    mesh=_mesh,
    scratch_types=[
        pltpu.VMEM_SHARED((NP, D), jnp.float32),
        pltpu.VMEM((CH,), jnp.int32),
        pltpu.VMEM((CH,), jnp.int32),
        pltpu.VMEM((CH,), jnp.int32),
        pltpu.VMEM((CH,), jnp.int32),
        pltpu.VMEM((CH, D), jnp.float32),
        pltpu.VMEM((CH, D), jnp.float32),
        pltpu.SemaphoreType.DMA,
        pltpu.SemaphoreType.DMA,
        pltpu.SemaphoreType.DMA,
        pltpu.SemaphoreType.DMA,
    ],
)
def _agg_kernel(src_hbm, dst_hbm, hs_hbm, out_hbm, acc, sidx0, sidx1,
                didx0, didx1, rows0, rows1, isem0, isem1, gsem0, gsem1):
    c = lax.axis_index("c")
    s = lax.axis_index("s")
    wid = s * NC + c
    sbufs = (sidx0, sidx1)
    dbufs = (didx0, didx1)
    rbufs = (rows0, rows1)
    isems = (isem0, isem1)
    gsems = (gsem0, gsem1)

    zero16 = jnp.zeros((16,), dtype=jnp.float32)

    @pl.loop(0, CH)
    def _(r):
        @pl.loop(0, D // 16)
        def _(j):
            rows0[r, pl.ds(j * 16, 16)] = zero16

    rbase = s * (NP // NS)
    for k in range(5):
        pltpu.sync_copy(rows0, acc.at[pl.ds(rbase + k * CH, CH)])

    plsc.subcore_barrier()

    nch = jnp.where(wid < ECH - (ECH // NW) * NW, ECH // NW + 1, ECH // NW)

    def _idx_start(chunk, b):
        base = (wid + chunk * NW) * CH
        pltpu.async_copy(src_hbm.at[pl.ds(base, CH)], sbufs[b], isems[b])
        pltpu.async_copy(dst_hbm.at[pl.ds(base, CH)], dbufs[b], isems[b])

    def _idx_wait(chunk, b):
        base = (wid + chunk * NW) * CH
        pltpu.make_async_copy(src_hbm.at[pl.ds(base, CH)],
                              sbufs[b], isems[b]).wait()
        pltpu.make_async_copy(dst_hbm.at[pl.ds(base, CH)],
                              dbufs[b], isems[b]).wait()

    def _gather_start(b):
        pltpu.async_copy(hs_hbm.at[sbufs[b]], rbufs[b], gsems[b])

    def _gather_wait(b):
        pltpu.make_async_copy(hs_hbm.at[sbufs[b]], rbufs[b], gsems[b]).wait()

    _idx_start(0, 0)
    _idx_start(1, 1)
    _idx_wait(0, 0)
    _gather_start(0)

    @pl.loop(0, (ECH // NW + 1 + 1) // 2 * 2, step=2)
    def _(k):
        for b in range(2):
            cur = k + b

            @pl.when(cur < nch)
            def _():
                nb = 1 - b

                @pl.when(cur + 1 < nch)
                def _():
                    _idx_wait(cur + 1, nb)
                    _gather_start(nb)

                _gather_wait(b)
                pltpu.sync_copy(rbufs[b], acc.at[dbufs[b]], add=True)

                @pl.when(cur + 2 < nch)
                def _():
                    _idx_start(cur + 2, b)

    plsc.subcore_barrier()

    for k in range(5):
        pltpu.sync_copy(acc.at[pl.ds(rbase + k * CH, CH)], rows0)
        pltpu.sync_copy(rows0, out_hbm.at[pl.ds(c * NP + rbase + k * CH, CH)])


def _fin_body(y_ref, hs_ref, dinv_ref, b_ref, gamma_ref, beta_ref, out_ref,
              acc_s, acc_q, scale_s, shift_s):
    p = pl.program_id(0)
    i = pl.program_id(1)
    t = (y_ref[0] + y_ref[1] + hs_ref[...]) * dinv_ref[...] + b_ref[...]
    z = jnp.where(t >= 0, t, NEG_SLOPE * t)

    @pl.when(jnp.logical_and(p == 0, i == 0))
    def _():
        acc_s[...] = jnp.zeros_like(acc_s)
        acc_q[...] = jnp.zeros_like(acc_q)

    @pl.when(p == 0)
    def _():
        acc_s[...] += jnp.sum(z, axis=0, keepdims=True)
        acc_q[...] += jnp.sum(z * z, axis=0, keepdims=True)
        out_ref[...] = z

    @pl.when(jnp.logical_and(p == 1, i == 0))
    def _():
        mean = acc_s[...] * (1.0 / N)
        var = acc_q[...] * (1.0 / N) - mean * mean
        g_rstd = gamma_ref[...] * lax.rsqrt(var + EPS)
        scale_s[...] = g_rstd
        shift_s[...] = beta_ref[...] - mean * g_rstd

    @pl.when(p == 1)
    def _():
        out_ref[...] = z * scale_s[...] + shift_s[...]


_fin_call = pl.pallas_call(
    _fin_body,
    grid=(2, GRID),
    in_specs=[
        pl.BlockSpec((NC, BM, D), lambda p, i: (0, i, 0)),
        pl.BlockSpec((BM, D), lambda p, i: (i, 0)),
        pl.BlockSpec((BM, 1), lambda p, i: (i, 0)),
        pl.BlockSpec((1, D), lambda p, i: (0, 0)),
        pl.BlockSpec((1, D), lambda p, i: (0, 0)),
        pl.BlockSpec((1, D), lambda p, i: (0, 0)),
    ],
    out_specs=pl.BlockSpec((BM, D), lambda p, i: (i, 0)),
    out_shape=jax.ShapeDtypeStruct((N, D), jnp.float32),
    scratch_shapes=[
        pltpu.VMEM((1, D), jnp.float32),
        pltpu.VMEM((1, D), jnp.float32),
        pltpu.VMEM((1, D), jnp.float32),
        pltpu.VMEM((1, D), jnp.float32),
    ],
)


def kernel(x, edge_index, W, b, gamma, beta):
    src = edge_index[0]
    dst = edge_index[1]
    npad = ECH2D * CH - E
    pad_dst = N + (jnp.arange(npad, dtype=jnp.int32) % (NP - N))
    dst2d = jnp.concatenate([dst, pad_dst]).reshape(ECH2D, CH)

    degp = _deg_kernel(dst2d).reshape(NC, NP, 1)
    hs, dinv = _hs_call(degp, x, W)
    y = _agg_kernel(src, dst, hs).reshape(NC, NP, D)
    return _fin_call(y, hs, dinv, b.reshape(1, D), gamma.reshape(1, D),
                     beta.reshape(1, D))


# speedup vs baseline: 2.4231x; 1.0736x over previous
import functools

import jax
import jax.numpy as jnp
from jax import lax
from jax.experimental import pallas as pl
from jax.experimental.pallas import tpu as pltpu
from jax.experimental.pallas import tpu_sc as plsc

N = 10000
E = 320000
D = 128
EPS = 1e-5
NEG_SLOPE = 0.01

NC, NS = 2, 16
NW = NC * NS
CH = 128
ECH = E // CH
NCH = 80
ECH2D = NW * NCH

BM = 400
GRID = N // BM
NP = 10240

_mesh = plsc.VectorSubcoreMesh(
    core_axis_name="c", subcore_axis_name="s", num_cores=NC, num_subcores=NS)


@functools.partial(
    pl.kernel,
    out_type=jax.ShapeDtypeStruct((NC * NP,), jnp.float32),
    mesh=_mesh,
    scratch_types=[
        pltpu.VMEM_SHARED((NP,), jnp.float32),
        pltpu.VMEM((NCH, CH), jnp.int32),
        pltpu.VMEM((CH,), jnp.float32),
        pltpu.VMEM((NP // NS,), jnp.float32),
    ],
)
def _deg_kernel(dst2d_hbm, out_hbm, acc, didx, ones, zbuf):
    c = lax.axis_index("c")
    s = lax.axis_index("s")
    wid = s * NC + c

    one16 = jnp.full((16,), 1.0, dtype=jnp.float32)
    zero16 = jnp.zeros((16,), dtype=jnp.float32)

    @pl.loop(0, CH // 16)
    def _(i):
        ones[pl.ds(i * 16, 16)] = one16

    @pl.loop(0, NP // NS // 16)
    def _(i):
        zbuf[pl.ds(i * 16, 16)] = zero16
    pltpu.sync_copy(zbuf, acc.at[pl.ds(s * (NP // NS), NP // NS)])

    pltpu.sync_copy(dst2d_hbm.at[pl.ds(wid * NCH, NCH)], didx)

    plsc.subcore_barrier()

    @pl.loop(0, NCH)
    def _(k):
        pltpu.sync_copy(ones, acc.at[didx.at[k]], add=True)

    plsc.subcore_barrier()

    pltpu.sync_copy(acc.at[pl.ds(s * (NP // NS), NP // NS)], zbuf)
    pltpu.sync_copy(zbuf, out_hbm.at[pl.ds(c * NP + s * (NP // NS), NP // NS)])


def _hs_body(deg_ref, x_ref, w_ref, hs_ref, dinv_ref):
    deg = 1.0 + deg_ref[0] + deg_ref[1]
    dinv = lax.rsqrt(deg)
    h = jnp.dot(x_ref[...], w_ref[...], preferred_element_type=jnp.float32)
    hs_ref[...] = h * dinv
    dinv_ref[...] = dinv


_hs_call = pl.pallas_call(
    _hs_body,
    grid=(GRID,),
    in_specs=[
        pl.BlockSpec((NC, BM, 1), lambda i: (0, i, 0)),
        pl.BlockSpec((BM, D), lambda i: (i, 0)),
        pl.BlockSpec((D, D), lambda i: (0, 0)),
    ],
    out_specs=[
        pl.BlockSpec((BM, D), lambda i: (i, 0)),
        pl.BlockSpec((BM, 1), lambda i: (i, 0)),
    ],
    out_shape=[
        jax.ShapeDtypeStruct((N, D), jnp.float32),
        jax.ShapeDtypeStruct((N, 1), jnp.float32),
    ],
)


@functools.partial(
    pl.kernel,
    out_type=jax.ShapeDtypeStruct((NC * NP, D), jnp.float32),
    mesh=_mesh,
    scratch_types=[
        pltpu.VMEM_SHARED((NP, D), jnp.float32),
        pltpu.VMEM((4, 2, CH), jnp.int32),
        pltpu.VMEM((CH, D), jnp.float32),
        pltpu.VMEM((CH, D), jnp.float32),
        pltpu.SemaphoreType.DMA,
        pltpu.SemaphoreType.DMA,
        pltpu.SemaphoreType.DMA,
        pltpu.SemaphoreType.DMA,
        pltpu.SemaphoreType.DMA,
        pltpu.SemaphoreType.DMA,
        pltpu.SemaphoreType.DMA,
        pltpu.SemaphoreType.DMA,
    ],
)
def _agg_kernel(srcdst_hbm, hs_hbm, out_hbm, acc, idx,
                rows0, rows1, i0, i1, i2, i3, g0, g1, s0, s1):
    c = lax.axis_index("c")
    s = lax.axis_index("s")
    wid = s * NC + c
    rbufs = (rows0, rows1)
    isems = (i0, i1, i2, i3)
    gsems = (g0, g1)
    ssems = (s0, s1)

    zero16 = jnp.zeros((16,), dtype=jnp.float32)

    @pl.loop(0, CH)
    def _(r):
        @pl.loop(0, D // 16)
        def _(j):
            rows0[r, pl.ds(j * 16, 16)] = zero16

    rbase = s * (NP // NS)
    for k in range(5):
        pltpu.sync_copy(rows0, acc.at[pl.ds(rbase + k * CH, CH)])

    plsc.subcore_barrier()

    cbase = wid * NCH

    def _idx_start(chunk, q):
        pltpu.async_copy(srcdst_hbm.at[cbase + chunk], idx.at[q], isems[q])

    def _idx_wait(chunk, q):
        pltpu.make_async_copy(srcdst_hbm.at[cbase + chunk],
                              idx.at[q], isems[q]).wait()

    def _gather_start(chunk, q, b):
        pltpu.async_copy(hs_hbm.at[idx.at[q, 0]], rbufs[b], gsems[b])

    def _gather_wait(chunk, q, b):
        pltpu.make_async_copy(hs_hbm.at[idx.at[q, 0]],
                              rbufs[b], gsems[b]).wait()

    def _scatter_start(chunk, q, b):
        pltpu.async_copy(rbufs[b], acc.at[idx.at[q, 1]], ssems[b], add=True)

    def _scatter_wait(chunk, q, b):
        pltpu.make_async_copy(rbufs[b], acc.at[idx.at[q, 1]],
                              ssems[b]).wait()

    def _steady(i, q, b):
        qm1 = (q + 3) % 4
        qm2 = (q + 2) % 4
        _idx_wait(i, q)
        _scatter_wait(i - 2, qm2, b)
        _gather_start(i, q, b)
        _gather_wait(i - 1, qm1, 1 - b)
        _scatter_start(i - 1, qm1, 1 - b)

        @pl.when(i + 2 < NCH)
        def _():
            _idx_start(i + 2, qm2)

    _idx_start(0, 0)
    _idx_start(1, 1)

    _idx_wait(0, 0)
    _gather_start(0, 0, 0)
    _idx_start(2, 2)

    _idx_wait(1, 1)
    _gather_start(1, 1, 1)
    _gather_wait(0, 0, 0)
    _scatter_start(0, 0, 0)
    _idx_start(3, 3)

    _steady(2, 2, 0)
    _steady(3, 3, 1)

    @pl.loop(0, (NCH - 4) // 4)
    def _(k):
        for j in range(4):
            _steady(4 + 4 * k + j, j, j % 2)

    _gather_wait(NCH - 1, (NCH - 1) % 4, (NCH - 1) % 2)
    _scatter_start(NCH - 1, (NCH - 1) % 4, (NCH - 1) % 2)
    _scatter_wait(NCH - 2, (NCH - 2) % 4, (NCH - 2) % 2)
    _scatter_wait(NCH - 1, (NCH - 1) % 4, (NCH - 1) % 2)

    plsc.subcore_barrier()

    for k in range(5):
        pltpu.sync_copy(acc.at[pl.ds(rbase + k * CH, CH)], rows0)
        pltpu.sync_copy(rows0, out_hbm.at[pl.ds(c * NP + rbase + k * CH, CH)])


def _fin_body(y_ref, hs_ref, dinv_ref, b_ref, gamma_ref, beta_ref, out_ref,
              acc_s, acc_q, scale_s, shift_s):
    p = pl.program_id(0)
    i = pl.program_id(1)
    t = (y_ref[0] + y_ref[1] + hs_ref[...]) * dinv_ref[...] + b_ref[...]
    z = jnp.where(t >= 0, t, NEG_SLOPE * t)

    @pl.when(jnp.logical_and(p == 0, i == 0))
    def _():
        acc_s[...] = jnp.zeros_like(acc_s)
        acc_q[...] = jnp.zeros_like(acc_q)

    @pl.when(p == 0)
    def _():
        acc_s[...] += jnp.sum(z, axis=0, keepdims=True)
        acc_q[...] += jnp.sum(z * z, axis=0, keepdims=True)
        out_ref[...] = z

    @pl.when(jnp.logical_and(p == 1, i == 0))
    def _():
        mean = acc_s[...] * (1.0 / N)
        var = acc_q[...] * (1.0 / N) - mean * mean
        g_rstd = gamma_ref[...] * lax.rsqrt(var + EPS)
        scale_s[...] = g_rstd
        shift_s[...] = beta_ref[...] - mean * g_rstd

    @pl.when(p == 1)
    def _():
        out_ref[...] = z * scale_s[...] + shift_s[...]


_fin_call = pl.pallas_call(
    _fin_body,
    grid=(2, GRID),
    in_specs=[
        pl.BlockSpec((NC, BM, D), lambda p, i: (0, i, 0)),
        pl.BlockSpec((BM, D), lambda p, i: (i, 0)),
        pl.BlockSpec((BM, 1), lambda p, i: (i, 0)),
        pl.BlockSpec((1, D), lambda p, i: (0, 0)),
        pl.BlockSpec((1, D), lambda p, i: (0, 0)),
        pl.BlockSpec((1, D), lambda p, i: (0, 0)),
    ],
    out_specs=pl.BlockSpec((BM, D), lambda p, i: (i, 0)),
    out_shape=jax.ShapeDtypeStruct((N, D), jnp.float32),
    scratch_shapes=[
        pltpu.VMEM((1, D), jnp.float32),
        pltpu.VMEM((1, D), jnp.float32),
        pltpu.VMEM((1, D), jnp.float32),
        pltpu.VMEM((1, D), jnp.float32),
    ],
)


def kernel(x, edge_index, W, b, gamma, beta):
    npad = ECH2D * CH - E
    ar = jnp.arange(npad, dtype=jnp.int32)
    srcp = jnp.concatenate([edge_index[0], (ar * 37) % N]).reshape(ECH2D, CH)
    dst2d = jnp.concatenate([edge_index[1], N + ar % (NP - N)]).reshape(ECH2D, CH)
    srcdst = jnp.stack([srcp, dst2d], axis=1)

    degp = _deg_kernel(dst2d).reshape(NC, NP, 1)
    hs, dinv = _hs_call(degp, x, W)
    y = _agg_kernel(srcdst, hs).reshape(NC, NP, D)
    return _fin_call(y, hs, dinv, b.reshape(1, D), gamma.reshape(1, D),
                     beta.reshape(1, D))


# speedup vs baseline: 2.5560x; 1.0549x over previous
import functools

import jax
import jax.numpy as jnp
from jax import lax
from jax.experimental import pallas as pl
from jax.experimental.pallas import tpu as pltpu
from jax.experimental.pallas import tpu_sc as plsc

N = 10000
E = 320000
D = 128
EPS = 1e-5
NEG_SLOPE = 0.01

NC, NS = 2, 16
NW = NC * NS
CH = 128
ECH = E // CH
NCH = 80
ECH2D = NW * NCH

BM = 400
GRID = N // BM
NP = 10240

_mesh = plsc.VectorSubcoreMesh(
    core_axis_name="c", subcore_axis_name="s", num_cores=NC, num_subcores=NS)


@functools.partial(
    pl.kernel,
    out_type=jax.ShapeDtypeStruct((NC, NP, D), jnp.float32),
    mesh=_mesh,
    scratch_types=[
        pltpu.VMEM_SHARED((NP,), jnp.float32),
        pltpu.VMEM((NCH, CH), jnp.int32),
        pltpu.VMEM((CH,), jnp.float32),
        pltpu.VMEM((NP // NS,), jnp.float32),
        pltpu.VMEM((CH, D), jnp.float32),
    ],
)
def _deg_kernel(dst2d_hbm, out_hbm, acc, didx, ones, zbuf, colbuf):
    c = lax.axis_index("c")
    s = lax.axis_index("s")
    wid = s * NC + c

    one16 = jnp.full((16,), 1.0, dtype=jnp.float32)
    zero16 = jnp.zeros((16,), dtype=jnp.float32)

    @pl.loop(0, CH // 16)
    def _(i):
        ones[pl.ds(i * 16, 16)] = one16

    @pl.loop(0, NP // NS // 16)
    def _(i):
        zbuf[pl.ds(i * 16, 16)] = zero16
    pltpu.sync_copy(zbuf, acc.at[pl.ds(s * (NP // NS), NP // NS)])

    pltpu.sync_copy(dst2d_hbm.at[pl.ds(wid * NCH, NCH)], didx)

    plsc.subcore_barrier()

    @pl.loop(0, NCH)
    def _(k):
        pltpu.sync_copy(ones, acc.at[didx.at[k]], add=True)

    plsc.subcore_barrier()

    pltpu.sync_copy(acc.at[pl.ds(s * (NP // NS), NP // NS)], zbuf)

    @pl.loop(0, 5)
    def _(k):
        @pl.loop(0, CH // 16)
        def _(i):
            v = zbuf[pl.ds(k * CH + i * 16, 16)]
            for j in range(16):
                colbuf[i * 16 + j, pl.ds(0, 16)] = jnp.full(
                    (16,), v[j], dtype=jnp.float32)
        pltpu.sync_copy(
            colbuf, out_hbm.at[c, pl.ds(s * (NP // NS) + k * CH, CH)])


def _hs_body(deg_ref, x_ref, w_ref, hs_ref, dinv_ref):
    deg = 1.0 + deg_ref[0, :, 0:1] + deg_ref[1, :, 0:1]
    dinv = lax.rsqrt(deg)
    h = jnp.dot(x_ref[...], w_ref[...], preferred_element_type=jnp.float32)
    hs_ref[...] = h * dinv
    dinv_ref[...] = jnp.broadcast_to(dinv, (BM, D))


_hs_call = pl.pallas_call(
    _hs_body,
    grid=(GRID,),
    in_specs=[
        pl.BlockSpec((NC, BM, D), lambda i: (0, i, 0)),
        pl.BlockSpec((BM, D), lambda i: (i, 0)),
        pl.BlockSpec((D, D), lambda i: (0, 0)),
    ],
    out_specs=[
        pl.BlockSpec((BM, D), lambda i: (i, 0)),
        pl.BlockSpec((BM, D), lambda i: (i, 0)),
    ],
    out_shape=[
        jax.ShapeDtypeStruct((N, D), jnp.float32),
        jax.ShapeDtypeStruct((N, D), jnp.float32),
    ],
)


@functools.partial(
    pl.kernel,
    out_type=jax.ShapeDtypeStruct((NC * NP, D), jnp.float32),
    mesh=_mesh,
    scratch_types=[
        pltpu.VMEM_SHARED((NP, D), jnp.float32),
        pltpu.VMEM((4, 2, CH), jnp.int32),
        pltpu.VMEM((CH, D), jnp.float32),
        pltpu.VMEM((CH, D), jnp.float32),
        pltpu.SemaphoreType.DMA,
        pltpu.SemaphoreType.DMA,
        pltpu.SemaphoreType.DMA,
        pltpu.SemaphoreType.DMA,
        pltpu.SemaphoreType.DMA,
        pltpu.SemaphoreType.DMA,
        pltpu.SemaphoreType.DMA,
        pltpu.SemaphoreType.DMA,
    ],
)
def _agg_kernel(srcdst_hbm, hs_hbm, out_hbm, acc, idx,
                rows0, rows1, i0, i1, i2, i3, g0, g1, s0, s1):
    c = lax.axis_index("c")
    s = lax.axis_index("s")
    wid = s * NC + c
    rbufs = (rows0, rows1)
    isems = (i0, i1, i2, i3)
    gsems = (g0, g1)
    ssems = (s0, s1)

    zero16 = jnp.zeros((16,), dtype=jnp.float32)

    @pl.loop(0, CH)
    def _(r):
        @pl.loop(0, D // 16)
        def _(j):
            rows0[r, pl.ds(j * 16, 16)] = zero16

    rbase = s * (NP // NS)
    for k in range(5):
        pltpu.sync_copy(rows0, acc.at[pl.ds(rbase + k * CH, CH)])

    plsc.subcore_barrier()

    cbase = wid * NCH

    def _idx_start(chunk, q):
        pltpu.async_copy(srcdst_hbm.at[cbase + chunk], idx.at[q], isems[q])

    def _idx_wait(chunk, q):
        pltpu.make_async_copy(srcdst_hbm.at[cbase + chunk],
                              idx.at[q], isems[q]).wait()

    def _gather_start(chunk, q, b):
        pltpu.async_copy(hs_hbm.at[idx.at[q, 0]], rbufs[b], gsems[b])

    def _gather_wait(chunk, q, b):
        pltpu.make_async_copy(hs_hbm.at[idx.at[q, 0]],
                              rbufs[b], gsems[b]).wait()

    def _scatter_start(chunk, q, b):
        pltpu.async_copy(rbufs[b], acc.at[idx.at[q, 1]], ssems[b], add=True)

    def _scatter_wait(chunk, q, b):
        pltpu.make_async_copy(rbufs[b], acc.at[idx.at[q, 1]],
                              ssems[b]).wait()

    def _steady(i, q, b):
        qm1 = (q + 3) % 4
        qm2 = (q + 2) % 4
        _idx_wait(i, q)
        _scatter_wait(i - 2, qm2, b)
        _gather_start(i, q, b)
        _gather_wait(i - 1, qm1, 1 - b)
        _scatter_start(i - 1, qm1, 1 - b)

        @pl.when(i + 2 < NCH)
        def _():
            _idx_start(i + 2, qm2)

    _idx_start(0, 0)
    _idx_start(1, 1)

    _idx_wait(0, 0)
    _gather_start(0, 0, 0)
    _idx_start(2, 2)

    _idx_wait(1, 1)
    _gather_start(1, 1, 1)
    _gather_wait(0, 0, 0)
    _scatter_start(0, 0, 0)
    _idx_start(3, 3)

    _steady(2, 2, 0)
    _steady(3, 3, 1)

    @pl.loop(0, (NCH - 4) // 4)
    def _(k):
        for j in range(4):
            _steady(4 + 4 * k + j, j, j % 2)

    _gather_wait(NCH - 1, (NCH - 1) % 4, (NCH - 1) % 2)
    _scatter_start(NCH - 1, (NCH - 1) % 4, (NCH - 1) % 2)
    _scatter_wait(NCH - 2, (NCH - 2) % 4, (NCH - 2) % 2)
    _scatter_wait(NCH - 1, (NCH - 1) % 4, (NCH - 1) % 2)

    plsc.subcore_barrier()

    for k in range(5):
        pltpu.sync_copy(acc.at[pl.ds(rbase + k * CH, CH)], rows0)
        pltpu.sync_copy(rows0, out_hbm.at[pl.ds(c * NP + rbase + k * CH, CH)])


def _fin_body(y_ref, hs_ref, dinv_ref, b_ref, gamma_ref, beta_ref, out_ref,
              z_s, acc_s, acc_q, scale_s, shift_s):
    p = pl.program_id(0)
    i = pl.program_id(1)

    @pl.when(p == 0)
    def _():
        t = (y_ref[0] + y_ref[1] + hs_ref[...]) * dinv_ref[...] + b_ref[...]
        z = jnp.where(t >= 0, t, NEG_SLOPE * t)
        z_s[pl.ds(i * BM, BM), :] = z

        @pl.when(i == 0)
        def _():
            acc_s[...] = jnp.zeros_like(acc_s)
            acc_q[...] = jnp.zeros_like(acc_q)

        acc_s[...] += jnp.sum(z, axis=0, keepdims=True)
        acc_q[...] += jnp.sum(z * z, axis=0, keepdims=True)

    @pl.when(jnp.logical_and(p == 1, i == 0))
    def _():
        mean = acc_s[...] * (1.0 / N)
        var = acc_q[...] * (1.0 / N) - mean * mean
        g_rstd = gamma_ref[...] * lax.rsqrt(var + EPS)
        scale_s[...] = g_rstd
        shift_s[...] = beta_ref[...] - mean * g_rstd

    @pl.when(p == 1)
    def _():
        out_ref[...] = z_s[pl.ds(i * BM, BM), :] * scale_s[...] + shift_s[...]


_fin_call = pl.pallas_call(
    _fin_body,
    grid=(2, GRID),
    in_specs=[
        pl.BlockSpec((NC, BM, D), lambda p, i: (0, i, 0)),
        pl.BlockSpec((BM, D), lambda p, i: (i, 0)),
        pl.BlockSpec((BM, D), lambda p, i: (i, 0)),
        pl.BlockSpec((1, D), lambda p, i: (0, 0)),
        pl.BlockSpec((1, D), lambda p, i: (0, 0)),
        pl.BlockSpec((1, D), lambda p, i: (0, 0)),
    ],
    out_specs=pl.BlockSpec((BM, D), lambda p, i: (i, 0)),
    out_shape=jax.ShapeDtypeStruct((N, D), jnp.float32),
    scratch_shapes=[
        pltpu.VMEM((N, D), jnp.float32),
        pltpu.VMEM((1, D), jnp.float32),
        pltpu.VMEM((1, D), jnp.float32),
        pltpu.VMEM((1, D), jnp.float32),
        pltpu.VMEM((1, D), jnp.float32),
    ],
)


def kernel(x, edge_index, W, b, gamma, beta):
    npad = ECH2D * CH - E
    ar = jnp.arange(npad, dtype=jnp.int32)
    srcp = jnp.concatenate([edge_index[0], (ar * 37) % N]).reshape(ECH2D, CH)
    dst2d = jnp.concatenate([edge_index[1], N + ar % (NP - N)]).reshape(ECH2D, CH)
    srcdst = jnp.stack([srcp, dst2d], axis=1)

    degp = _deg_kernel(dst2d)
    hs, dinv = _hs_call(degp, x, W)
    y = _agg_kernel(srcdst, hs).reshape(NC, NP, D)
    return _fin_call(y, hs, dinv, b.reshape(1, D), gamma.reshape(1, D),
                     beta.reshape(1, D))


# speedup vs baseline: 2.5772x; 1.0083x over previous
import functools

import jax
import jax.numpy as jnp
from jax import lax
from jax.experimental import pallas as pl
from jax.experimental.pallas import tpu as pltpu
from jax.experimental.pallas import tpu_sc as plsc

N = 10000
E = 320000
D = 128
EPS = 1e-5
NEG_SLOPE = 0.01

NC, NS = 2, 16
NW = NC * NS
CH = 128
ECH = E // CH
NCH = 80
ECH2D = NW * NCH

BM = 400
GRID = N // BM
NP = 10240

_mesh = plsc.VectorSubcoreMesh(
    core_axis_name="c", subcore_axis_name="s", num_cores=NC, num_subcores=NS)


@functools.partial(
    pl.kernel,
    out_type=jax.ShapeDtypeStruct((NC, NP, D), jnp.float32),
    mesh=_mesh,
    scratch_types=[
        pltpu.VMEM_SHARED((NP,), jnp.float32),
        pltpu.VMEM((NCH, CH), jnp.int32),
        pltpu.VMEM((CH,), jnp.float32),
        pltpu.VMEM((NP // NS,), jnp.float32),
        pltpu.VMEM((CH, D), jnp.float32),
    ],
)
def _deg_kernel(dst2d_hbm, out_hbm, acc, didx, ones, zbuf, colbuf):
    c = lax.axis_index("c")
    s = lax.axis_index("s")
    wid = s * NC + c

    one16 = jnp.full((16,), 1.0, dtype=jnp.float32)
    zero16 = jnp.zeros((16,), dtype=jnp.float32)

    @pl.loop(0, CH // 16)
    def _(i):
        ones[pl.ds(i * 16, 16)] = one16

    @pl.loop(0, NP // NS // 16)
    def _(i):
        zbuf[pl.ds(i * 16, 16)] = zero16
    pltpu.sync_copy(zbuf, acc.at[pl.ds(s * (NP // NS), NP // NS)])

    pltpu.sync_copy(dst2d_hbm.at[pl.ds(wid * NCH, NCH)], didx)

    plsc.subcore_barrier()

    @pl.loop(0, NCH)
    def _(k):
        pltpu.sync_copy(ones, acc.at[didx.at[k]], add=True)

    plsc.subcore_barrier()

    pltpu.sync_copy(acc.at[pl.ds(s * (NP // NS), NP // NS)], zbuf)

    @pl.loop(0, 5)
    def _(k):
        @pl.loop(0, CH // 16)
        def _(i):
            v = zbuf[pl.ds(k * CH + i * 16, 16)]
            for j in range(16):
                colbuf[i * 16 + j, pl.ds(0, 16)] = jnp.full(
                    (16,), v[j], dtype=jnp.float32)
        pltpu.sync_copy(
            colbuf, out_hbm.at[c, pl.ds(s * (NP // NS) + k * CH, CH)])


def _hs_body(deg_ref, x_ref, w_ref, hs_ref, dinv_ref):
    deg = 1.0 + deg_ref[0, :, 0:1] + deg_ref[1, :, 0:1]
    dinv = lax.rsqrt(deg)
    h = jnp.dot(x_ref[...], w_ref[...], preferred_element_type=jnp.float32)
    hs_ref[...] = h * dinv
    dinv_ref[...] = jnp.broadcast_to(dinv, (BM, D))


_hs_call = pl.pallas_call(
    _hs_body,
    grid=(GRID,),
    in_specs=[
        pl.BlockSpec((NC, BM, D), lambda i: (0, i, 0)),
        pl.BlockSpec((BM, D), lambda i: (i, 0)),
        pl.BlockSpec((D, D), lambda i: (0, 0)),
    ],
    out_specs=[
        pl.BlockSpec((BM, D), lambda i: (i, 0)),
        pl.BlockSpec((BM, D), lambda i: (i, 0)),
    ],
    out_shape=[
        jax.ShapeDtypeStruct((N, D), jnp.float32),
        jax.ShapeDtypeStruct((N, D), jnp.float32),
    ],
)


@functools.partial(
    pl.kernel,
    out_type=jax.ShapeDtypeStruct((NC * NP, D), jnp.float32),
    mesh=_mesh,
    scratch_types=[
        pltpu.VMEM_SHARED((NP, D), jnp.float32),
        pltpu.VMEM((4, 2, CH), jnp.int32),
        pltpu.VMEM((CH, D), jnp.float32),
        pltpu.VMEM((CH, D), jnp.float32),
        pltpu.SemaphoreType.DMA,
        pltpu.SemaphoreType.DMA,
        pltpu.SemaphoreType.DMA,
        pltpu.SemaphoreType.DMA,
        pltpu.SemaphoreType.DMA,
        pltpu.SemaphoreType.DMA,
        pltpu.SemaphoreType.DMA,
        pltpu.SemaphoreType.DMA,
    ],
)
def _agg_kernel(src_hbm, dst_hbm, hs_hbm, out_hbm, acc, idx,
                rows0, rows1, i0, i1, i2, i3, g0, g1, s0, s1):
    c = lax.axis_index("c")
    s = lax.axis_index("s")
    wid = s * NC + c
    rbufs = (rows0, rows1)
    isems = (i0, i1, i2, i3)
    gsems = (g0, g1)
    ssems = (s0, s1)

    zero16 = jnp.zeros((16,), dtype=jnp.float32)

    @pl.loop(0, CH)
    def _(r):
        @pl.loop(0, D // 16)
        def _(j):
            rows0[r, pl.ds(j * 16, 16)] = zero16

    rbase = s * (NP // NS)
    for k in range(5):
        pltpu.sync_copy(rows0, acc.at[pl.ds(rbase + k * CH, CH)])

    plsc.subcore_barrier()

    ebase = wid * NCH * CH

    def _idx_start(chunk, q):
        pltpu.async_copy(src_hbm.at[pl.ds(ebase + chunk * CH, CH)],
                         idx.at[q, 0], isems[q])
        pltpu.async_copy(dst_hbm.at[pl.ds(ebase + chunk * CH, CH)],
                         idx.at[q, 1], isems[q])

    def _idx_wait(chunk, q):
        pltpu.make_async_copy(src_hbm.at[pl.ds(ebase + chunk * CH, CH)],
                              idx.at[q, 0], isems[q]).wait()
        pltpu.make_async_copy(dst_hbm.at[pl.ds(ebase + chunk * CH, CH)],
                              idx.at[q, 1], isems[q]).wait()

    def _gather_start(chunk, q, b):
        pltpu.async_copy(hs_hbm.at[idx.at[q, 0]], rbufs[b], gsems[b])

    def _gather_wait(chunk, q, b):
        pltpu.make_async_copy(hs_hbm.at[idx.at[q, 0]],
                              rbufs[b], gsems[b]).wait()

    def _scatter_start(chunk, q, b):
        pltpu.async_copy(rbufs[b], acc.at[idx.at[q, 1]], ssems[b], add=True)

    def _scatter_wait(chunk, q, b):
        pltpu.make_async_copy(rbufs[b], acc.at[idx.at[q, 1]],
                              ssems[b]).wait()

    def _steady(i, q, b):
        qm1 = (q + 3) % 4
        qm2 = (q + 2) % 4
        _idx_wait(i, q)
        _scatter_wait(i - 2, qm2, b)
        _gather_start(i, q, b)
        _gather_wait(i - 1, qm1, 1 - b)
        _scatter_start(i - 1, qm1, 1 - b)

        @pl.when(i + 2 < NCH)
        def _():
            _idx_start(i + 2, qm2)

    _idx_start(0, 0)
    _idx_start(1, 1)

    _idx_wait(0, 0)
    _gather_start(0, 0, 0)
    _idx_start(2, 2)

    _idx_wait(1, 1)
    _gather_start(1, 1, 1)
    _gather_wait(0, 0, 0)
    _scatter_start(0, 0, 0)
    _idx_start(3, 3)

    _steady(2, 2, 0)
    _steady(3, 3, 1)

    @pl.loop(0, (NCH - 4) // 4)
    def _(k):
        for j in range(4):
            _steady(4 + 4 * k + j, j, j % 2)

    _gather_wait(NCH - 1, (NCH - 1) % 4, (NCH - 1) % 2)
    _scatter_start(NCH - 1, (NCH - 1) % 4, (NCH - 1) % 2)
    _scatter_wait(NCH - 2, (NCH - 2) % 4, (NCH - 2) % 2)
    _scatter_wait(NCH - 1, (NCH - 1) % 4, (NCH - 1) % 2)

    plsc.subcore_barrier()

    for k in range(5):
        pltpu.sync_copy(acc.at[pl.ds(rbase + k * CH, CH)], rows0)
        pltpu.sync_copy(rows0, out_hbm.at[pl.ds(c * NP + rbase + k * CH, CH)])


def _fin_body(y_ref, hs_ref, dinv_ref, b_ref, gamma_ref, beta_ref, out_ref,
              z_s, acc_s, acc_q, scale_s, shift_s):
    p = pl.program_id(0)
    i = pl.program_id(1)

    @pl.when(p == 0)
    def _():
        t = (y_ref[0] + y_ref[1] + hs_ref[...]) * dinv_ref[...] + b_ref[...]
        z = jnp.where(t >= 0, t, NEG_SLOPE * t)
        z_s[pl.ds(i * BM, BM), :] = z

        @pl.when(i == 0)
        def _():
            acc_s[...] = jnp.zeros_like(acc_s)
            acc_q[...] = jnp.zeros_like(acc_q)

        acc_s[...] += jnp.sum(z, axis=0, keepdims=True)
        acc_q[...] += jnp.sum(z * z, axis=0, keepdims=True)

    @pl.when(jnp.logical_and(p == 1, i == 0))
    def _():
        mean = acc_s[...] * (1.0 / N)
        var = acc_q[...] * (1.0 / N) - mean * mean
        g_rstd = gamma_ref[...] * lax.rsqrt(var + EPS)
        scale_s[...] = g_rstd
        shift_s[...] = beta_ref[...] - mean * g_rstd

    @pl.when(p == 1)
    def _():
        out_ref[...] = z_s[pl.ds(i * BM, BM), :] * scale_s[...] + shift_s[...]


_fin_call = pl.pallas_call(
    _fin_body,
    grid=(2, GRID),
    in_specs=[
        pl.BlockSpec((NC, BM, D), lambda p, i: (0, i, 0)),
        pl.BlockSpec((BM, D), lambda p, i: (i, 0)),
        pl.BlockSpec((BM, D), lambda p, i: (i, 0)),
        pl.BlockSpec((1, D), lambda p, i: (0, 0)),
        pl.BlockSpec((1, D), lambda p, i: (0, 0)),
        pl.BlockSpec((1, D), lambda p, i: (0, 0)),
    ],
    out_specs=pl.BlockSpec((BM, D), lambda p, i: (jnp.where(p == 0, 0, i), 0)),
    out_shape=jax.ShapeDtypeStruct((N, D), jnp.float32),
    scratch_shapes=[
        pltpu.VMEM((N, D), jnp.float32),
        pltpu.VMEM((1, D), jnp.float32),
        pltpu.VMEM((1, D), jnp.float32),
        pltpu.VMEM((1, D), jnp.float32),
        pltpu.VMEM((1, D), jnp.float32),
    ],
)


def kernel(x, edge_index, W, b, gamma, beta):
    npad = ECH2D * CH - E
    ar = jnp.arange(npad, dtype=jnp.int32)
    srcp = jnp.concatenate([edge_index[0], (ar * 37) % N])
    dstp = jnp.concatenate([edge_index[1], N + ar % (NP - N)])
    dst2d = dstp.reshape(ECH2D, CH)

    degp = _deg_kernel(dst2d)
    hs, dinv = _hs_call(degp, x, W)
    y = _agg_kernel(srcp, dstp, hs).reshape(NC, NP, D)
    return _fin_call(y, hs, dinv, b.reshape(1, D), gamma.reshape(1, D),
                     beta.reshape(1, D))


# speedup vs baseline: 2.7505x; 1.0673x over previous
import functools

import jax
import jax.numpy as jnp
from jax import lax
from jax.experimental import pallas as pl
from jax.experimental.pallas import tpu as pltpu
from jax.experimental.pallas import tpu_sc as plsc

N = 10000
E = 320000
D = 128
EPS = 1e-5
NEG_SLOPE = 0.01

NC, NS = 2, 16
NW = NC * NS
CH = 128
ECH = E // CH
NCH = 80
ECH2D = NW * NCH

BM = 512
GRID = (N + BM - 1) // BM
NP = 10240

_mesh = plsc.VectorSubcoreMesh(
    core_axis_name="c", subcore_axis_name="s", num_cores=NC, num_subcores=NS)


@functools.partial(
    pl.kernel,
    out_type=jax.ShapeDtypeStruct((NC * NP,), jnp.float32),
    mesh=_mesh,
    scratch_types=[
        pltpu.VMEM_SHARED((NP,), jnp.float32),
        pltpu.VMEM((NCH, CH), jnp.int32),
        pltpu.VMEM((CH,), jnp.float32),
        pltpu.VMEM((NP // NS,), jnp.float32),
    ],
)
def _deg_kernel(dst2d_hbm, out_hbm, acc, didx, ones, zbuf):
    c = lax.axis_index("c")
    s = lax.axis_index("s")
    wid = s * NC + c

    one16 = jnp.full((16,), 1.0, dtype=jnp.float32)
    zero16 = jnp.zeros((16,), dtype=jnp.float32)

    @pl.loop(0, CH // 16)
    def _(i):
        ones[pl.ds(i * 16, 16)] = one16

    @pl.loop(0, NP // NS // 16)
    def _(i):
        zbuf[pl.ds(i * 16, 16)] = zero16
    pltpu.sync_copy(zbuf, acc.at[pl.ds(s * (NP // NS), NP // NS)])

    pltpu.sync_copy(dst2d_hbm.at[pl.ds(wid * NCH, NCH)], didx)

    plsc.subcore_barrier()

    @pl.loop(0, NCH)
    def _(k):
        pltpu.sync_copy(ones, acc.at[didx.at[k]], add=True)

    plsc.subcore_barrier()

    pltpu.sync_copy(acc.at[pl.ds(s * (NP // NS), NP // NS)], zbuf)
    pltpu.sync_copy(zbuf, out_hbm.at[pl.ds(c * NP + s * (NP // NS), NP // NS)])


def _dinv_col(deg_ref):
    deg = 1.0 + deg_ref[0:1, :] + deg_ref[1:2, :]
    return jnp.reshape(lax.rsqrt(deg), (BM, 1))


def _hs_body(deg_ref, x_ref, w_ref, hs_ref):
    h = jnp.dot(x_ref[...], w_ref[...], preferred_element_type=jnp.float32)
    hs_ref[...] = h * _dinv_col(deg_ref)


_hs_call = pl.pallas_call(
    _hs_body,
    grid=(GRID,),
    in_specs=[
        pl.BlockSpec((NC, BM), lambda i: (0, i)),
        pl.BlockSpec((BM, D), lambda i: (i, 0)),
        pl.BlockSpec((D, D), lambda i: (0, 0)),
    ],
    out_specs=pl.BlockSpec((BM, D), lambda i: (i, 0)),
    out_shape=jax.ShapeDtypeStruct((N, D), jnp.float32),
)


@functools.partial(
    pl.kernel,
    out_type=jax.ShapeDtypeStruct((NC * NP, D), jnp.float32),
    mesh=_mesh,
    scratch_types=[
        pltpu.VMEM_SHARED((NP, D), jnp.float32),
        pltpu.VMEM((4, 2, CH), jnp.int32),
        pltpu.VMEM((CH, D), jnp.float32),
        pltpu.VMEM((CH, D), jnp.float32),
        pltpu.SemaphoreType.DMA,
        pltpu.SemaphoreType.DMA,
        pltpu.SemaphoreType.DMA,
        pltpu.SemaphoreType.DMA,
        pltpu.SemaphoreType.DMA,
        pltpu.SemaphoreType.DMA,
        pltpu.SemaphoreType.DMA,
        pltpu.SemaphoreType.DMA,
    ],
)
def _agg_kernel(src_hbm, dst_hbm, hs_hbm, out_hbm, acc, idx,
                rows0, rows1, i0, i1, i2, i3, g0, g1, s0, s1):
    c = lax.axis_index("c")
    s = lax.axis_index("s")
    wid = s * NC + c
    rbufs = (rows0, rows1)
    isems = (i0, i1, i2, i3)
    gsems = (g0, g1)
    ssems = (s0, s1)

    zero16 = jnp.zeros((16,), dtype=jnp.float32)

    @pl.loop(0, CH)
    def _(r):
        @pl.loop(0, D // 16)
        def _(j):
            rows0[r, pl.ds(j * 16, 16)] = zero16

    rbase = s * (NP // NS)
    for k in range(5):
        pltpu.sync_copy(rows0, acc.at[pl.ds(rbase + k * CH, CH)])

    plsc.subcore_barrier()

    ebase = wid * NCH * CH

    def _idx_start(chunk, q):
        pltpu.async_copy(src_hbm.at[pl.ds(ebase + chunk * CH, CH)],
                         idx.at[q, 0], isems[q])
        pltpu.async_copy(dst_hbm.at[pl.ds(ebase + chunk * CH, CH)],
                         idx.at[q, 1], isems[q])

    def _idx_wait(chunk, q):
        pltpu.make_async_copy(src_hbm.at[pl.ds(ebase + chunk * CH, CH)],
                              idx.at[q, 0], isems[q]).wait()
        pltpu.make_async_copy(dst_hbm.at[pl.ds(ebase + chunk * CH, CH)],
                              idx.at[q, 1], isems[q]).wait()

    def _gather_start(chunk, q, b):
        pltpu.async_copy(hs_hbm.at[idx.at[q, 0]], rbufs[b], gsems[b])

    def _gather_wait(chunk, q, b):
        pltpu.make_async_copy(hs_hbm.at[idx.at[q, 0]],
                              rbufs[b], gsems[b]).wait()

    def _scatter_start(chunk, q, b):
        pltpu.async_copy(rbufs[b], acc.at[idx.at[q, 1]], ssems[b], add=True)

    def _scatter_wait(chunk, q, b):
        pltpu.make_async_copy(rbufs[b], acc.at[idx.at[q, 1]],
                              ssems[b]).wait()

    def _steady(i, q, b):
        qm1 = (q + 3) % 4
        qm2 = (q + 2) % 4
        _idx_wait(i, q)
        _scatter_wait(i - 2, qm2, b)
        _gather_start(i, q, b)
        _gather_wait(i - 1, qm1, 1 - b)
        _scatter_start(i - 1, qm1, 1 - b)

        @pl.when(i + 2 < NCH)
        def _():
            _idx_start(i + 2, qm2)

    _idx_start(0, 0)
    _idx_start(1, 1)

    _idx_wait(0, 0)
    _gather_start(0, 0, 0)
    _idx_start(2, 2)

    _idx_wait(1, 1)
    _gather_start(1, 1, 1)
    _gather_wait(0, 0, 0)
    _scatter_start(0, 0, 0)
    _idx_start(3, 3)

    _steady(2, 2, 0)
    _steady(3, 3, 1)

    @pl.loop(0, (NCH - 4) // 4)
    def _(k):
        for j in range(4):
            _steady(4 + 4 * k + j, j, j % 2)

    _gather_wait(NCH - 1, (NCH - 1) % 4, (NCH - 1) % 2)
    _scatter_start(NCH - 1, (NCH - 1) % 4, (NCH - 1) % 2)
    _scatter_wait(NCH - 2, (NCH - 2) % 4, (NCH - 2) % 2)
    _scatter_wait(NCH - 1, (NCH - 1) % 4, (NCH - 1) % 2)

    plsc.subcore_barrier()

    for k in range(5):
        pltpu.sync_copy(acc.at[pl.ds(rbase + k * CH, CH)], rows0)
        pltpu.sync_copy(rows0, out_hbm.at[pl.ds(c * NP + rbase + k * CH, CH)])


def _fin_body(y_ref, hs_ref, deg_ref, b_ref, gamma_ref, beta_ref, out_ref,
              z_s, acc_s, acc_q, scale_s, shift_s):
    p = pl.program_id(0)
    i = pl.program_id(1)

    @pl.when(p == 0)
    def _():
        t = ((y_ref[0] + y_ref[1] + hs_ref[...]) * _dinv_col(deg_ref)
             + b_ref[...])
        z = jnp.where(t >= 0, t, NEG_SLOPE * t)
        z_s[pl.ds(i * BM, BM), :] = z

        @pl.when(i == 0)
        def _():
            acc_s[...] = jnp.zeros_like(acc_s)
            acc_q[...] = jnp.zeros_like(acc_q)

        valid = (i * BM + lax.iota(jnp.int32, BM)[:, None]) < N
        zm = jnp.where(valid, z, 0.0)
        acc_s[...] += jnp.sum(zm, axis=0, keepdims=True)
        acc_q[...] += jnp.sum(zm * zm, axis=0, keepdims=True)

    @pl.when(jnp.logical_and(p == 1, i == 0))
    def _():
        mean = acc_s[...] * (1.0 / N)
        var = acc_q[...] * (1.0 / N) - mean * mean
        g_rstd = gamma_ref[...] * lax.rsqrt(var + EPS)
        scale_s[...] = g_rstd
        shift_s[...] = beta_ref[...] - mean * g_rstd

    @pl.when(p == 1)
    def _():
        out_ref[...] = z_s[pl.ds(i * BM, BM), :] * scale_s[...] + shift_s[...]


_fin_call = pl.pallas_call(
    _fin_body,
    grid=(2, GRID),
    in_specs=[
        pl.BlockSpec((NC, BM, D), lambda p, i: (0, i, 0)),
        pl.BlockSpec((BM, D), lambda p, i: (i, 0)),
        pl.BlockSpec((NC, BM), lambda p, i: (0, i)),
        pl.BlockSpec((1, D), lambda p, i: (0, 0)),
        pl.BlockSpec((1, D), lambda p, i: (0, 0)),
        pl.BlockSpec((1, D), lambda p, i: (0, 0)),
    ],
    out_specs=pl.BlockSpec((BM, D), lambda p, i: (jnp.where(p == 0, 0, i), 0)),
    out_shape=jax.ShapeDtypeStruct((N, D), jnp.float32),
    scratch_shapes=[
        pltpu.VMEM((GRID * BM, D), jnp.float32),
        pltpu.VMEM((1, D), jnp.float32),
        pltpu.VMEM((1, D), jnp.float32),
        pltpu.VMEM((1, D), jnp.float32),
        pltpu.VMEM((1, D), jnp.float32),
    ],
)


def kernel(x, edge_index, W, b, gamma, beta):
    npad = ECH2D * CH - E
    ar = jnp.arange(npad, dtype=jnp.int32)
    srcp = jnp.concatenate([edge_index[0], (ar * 37) % N])
    dstp = jnp.concatenate([edge_index[1], N + ar % (NP - N)])
    dst2d = dstp.reshape(ECH2D, CH)

    degp = _deg_kernel(dst2d).reshape(NC, NP)
    hs = _hs_call(degp, x, W)
    y = _agg_kernel(srcp, dstp, hs).reshape(NC, NP, D)
    return _fin_call(y, hs, degp, b.reshape(1, D), gamma.reshape(1, D),
                     beta.reshape(1, D))


# speedup vs baseline: 2.9912x; 1.0875x over previous
import functools

import jax
import jax.numpy as jnp
from jax import lax
from jax.experimental import pallas as pl
from jax.experimental.pallas import tpu as pltpu
from jax.experimental.pallas import tpu_sc as plsc

N = 10000
E = 320000
D = 128
EPS = 1e-5
NEG_SLOPE = 0.01

NC, NS = 2, 16
NW = NC * NS
CH = 128
ECH = E // CH
NCH = 80
ECH2D = NW * NCH

BM = 1024
GRID = (N + BM - 1) // BM
NP = 10240

_mesh = plsc.VectorSubcoreMesh(
    core_axis_name="c", subcore_axis_name="s", num_cores=NC, num_subcores=NS)


@functools.partial(
    pl.kernel,
    out_type=jax.ShapeDtypeStruct((NC * NP,), jnp.float32),
    mesh=_mesh,
    scratch_types=[
        pltpu.VMEM_SHARED((NP,), jnp.float32),
        pltpu.VMEM((NCH, CH), jnp.int32),
        pltpu.VMEM((CH,), jnp.float32),
        pltpu.VMEM((NP // NS,), jnp.float32),
    ],
)
def _deg_kernel(dst2d_hbm, out_hbm, acc, didx, ones, zbuf):
    c = lax.axis_index("c")
    s = lax.axis_index("s")
    wid = s * NC + c

    one16 = jnp.full((16,), 1.0, dtype=jnp.float32)
    zero16 = jnp.zeros((16,), dtype=jnp.float32)

    @pl.loop(0, CH // 16)
    def _(i):
        ones[pl.ds(i * 16, 16)] = one16

    @pl.loop(0, NP // NS // 16)
    def _(i):
        zbuf[pl.ds(i * 16, 16)] = zero16
    pltpu.sync_copy(zbuf, acc.at[pl.ds(s * (NP // NS), NP // NS)])

    pltpu.sync_copy(dst2d_hbm.at[pl.ds(wid * NCH, NCH)], didx)

    plsc.subcore_barrier()

    @pl.loop(0, NCH)
    def _(k):
        pltpu.sync_copy(ones, acc.at[didx.at[k]], add=True)

    plsc.subcore_barrier()

    pltpu.sync_copy(acc.at[pl.ds(s * (NP // NS), NP // NS)], zbuf)
    pltpu.sync_copy(zbuf, out_hbm.at[pl.ds(c * NP + s * (NP // NS), NP // NS)])


def _dinv_col(deg_ref):
    deg = 1.0 + deg_ref[0:1, :] + deg_ref[1:2, :]
    return jnp.reshape(lax.rsqrt(deg), (BM, 1))


def _hs_body(deg_ref, x_ref, w_ref, hs_ref):
    h = jnp.dot(x_ref[...], w_ref[...], preferred_element_type=jnp.float32)
    hs_ref[...] = h * _dinv_col(deg_ref)


_hs_call = pl.pallas_call(
    _hs_body,
    grid=(GRID,),
    in_specs=[
        pl.BlockSpec((NC, BM), lambda i: (0, i)),
        pl.BlockSpec((BM, D), lambda i: (i, 0)),
        pl.BlockSpec((D, D), lambda i: (0, 0)),
    ],
    out_specs=pl.BlockSpec((BM, D), lambda i: (i, 0)),
    out_shape=jax.ShapeDtypeStruct((N, D), jnp.float32),
)


@functools.partial(
    pl.kernel,
    out_type=jax.ShapeDtypeStruct((NC * NP, D), jnp.float32),
    mesh=_mesh,
    scratch_types=[
        pltpu.VMEM_SHARED((NP, D), jnp.float32),
        pltpu.VMEM((4, 2, CH), jnp.int32),
        pltpu.VMEM((CH, D), jnp.float32),
        pltpu.VMEM((CH, D), jnp.float32),
        pltpu.SemaphoreType.DMA,
        pltpu.SemaphoreType.DMA,
        pltpu.SemaphoreType.DMA,
        pltpu.SemaphoreType.DMA,
        pltpu.SemaphoreType.DMA,
        pltpu.SemaphoreType.DMA,
        pltpu.SemaphoreType.DMA,
        pltpu.SemaphoreType.DMA,
    ],
)
def _agg_kernel(src_hbm, dst_hbm, hs_hbm, out_hbm, acc, idx,
                rows0, rows1, i0, i1, i2, i3, g0, g1, s0, s1):
    c = lax.axis_index("c")
    s = lax.axis_index("s")
    wid = s * NC + c
    rbufs = (rows0, rows1)
    isems = (i0, i1, i2, i3)
    gsems = (g0, g1)
    ssems = (s0, s1)

    zero16 = jnp.zeros((16,), dtype=jnp.float32)

    @pl.loop(0, CH)
    def _(r):
        @pl.loop(0, D // 16)
        def _(j):
            rows0[r, pl.ds(j * 16, 16)] = zero16

    rbase = s * (NP // NS)
    for k in range(5):
        pltpu.sync_copy(rows0, acc.at[pl.ds(rbase + k * CH, CH)])

    plsc.subcore_barrier()

    ebase = wid * NCH * CH

    def _idx_start(chunk, q):
        pltpu.async_copy(src_hbm.at[pl.ds(ebase + chunk * CH, CH)],
                         idx.at[q, 0], isems[q])
        pltpu.async_copy(dst_hbm.at[pl.ds(ebase + chunk * CH, CH)],
                         idx.at[q, 1], isems[q])

    def _idx_wait(chunk, q):
        pltpu.make_async_copy(src_hbm.at[pl.ds(ebase + chunk * CH, CH)],
                              idx.at[q, 0], isems[q]).wait()
        pltpu.make_async_copy(dst_hbm.at[pl.ds(ebase + chunk * CH, CH)],
                              idx.at[q, 1], isems[q]).wait()

    def _gather_start(chunk, q, b):
        pltpu.async_copy(hs_hbm.at[idx.at[q, 0]], rbufs[b], gsems[b])

    def _gather_wait(chunk, q, b):
        pltpu.make_async_copy(hs_hbm.at[idx.at[q, 0]],
                              rbufs[b], gsems[b]).wait()

    def _scatter_start(chunk, q, b):
        pltpu.async_copy(rbufs[b], acc.at[idx.at[q, 1]], ssems[b], add=True)

    def _scatter_wait(chunk, q, b):
        pltpu.make_async_copy(rbufs[b], acc.at[idx.at[q, 1]],
                              ssems[b]).wait()

    def _steady(i, q, b):
        qm1 = (q + 3) % 4
        qm2 = (q + 2) % 4
        _idx_wait(i, q)
        _scatter_wait(i - 2, qm2, b)
        _gather_start(i, q, b)
        _gather_wait(i - 1, qm1, 1 - b)
        _scatter_start(i - 1, qm1, 1 - b)

        @pl.when(i + 2 < NCH)
        def _():
            _idx_start(i + 2, qm2)

    _idx_start(0, 0)
    _idx_start(1, 1)

    _idx_wait(0, 0)
    _gather_start(0, 0, 0)
    _idx_start(2, 2)

    _idx_wait(1, 1)
    _gather_start(1, 1, 1)
    _gather_wait(0, 0, 0)
    _scatter_start(0, 0, 0)
    _idx_start(3, 3)

    _steady(2, 2, 0)
    _steady(3, 3, 1)

    @pl.loop(0, (NCH - 4) // 4)
    def _(k):
        for j in range(4):
            _steady(4 + 4 * k + j, j, j % 2)

    _gather_wait(NCH - 1, (NCH - 1) % 4, (NCH - 1) % 2)
    _scatter_start(NCH - 1, (NCH - 1) % 4, (NCH - 1) % 2)
    _scatter_wait(NCH - 2, (NCH - 2) % 4, (NCH - 2) % 2)
    _scatter_wait(NCH - 1, (NCH - 1) % 4, (NCH - 1) % 2)

    plsc.subcore_barrier()

    for k in range(5):
        pltpu.sync_copy(acc.at[pl.ds(rbase + k * CH, CH)], rows0)
        pltpu.sync_copy(rows0, out_hbm.at[pl.ds(c * NP + rbase + k * CH, CH)])


def _fin_body(y_ref, hs_ref, deg_ref, b_ref, gamma_ref, beta_ref, out_ref,
              z_s, acc_s, acc_q, scale_s, shift_s):
    p = pl.program_id(0)
    i = pl.program_id(1)

    @pl.when(p == 0)
    def _():
        t = ((y_ref[0] + y_ref[1] + hs_ref[...]) * _dinv_col(deg_ref)
             + b_ref[...])
        z = jnp.where(t >= 0, t, NEG_SLOPE * t)
        z_s[pl.ds(i * BM, BM), :] = z

        @pl.when(i == 0)
        def _():
            acc_s[...] = jnp.zeros_like(acc_s)
            acc_q[...] = jnp.zeros_like(acc_q)

        valid = (i * BM + lax.iota(jnp.int32, BM)[:, None]) < N
        zm = jnp.where(valid, z, 0.0)
        acc_s[...] += jnp.sum(zm, axis=0, keepdims=True)
        acc_q[...] += jnp.sum(zm * zm, axis=0, keepdims=True)

    @pl.when(jnp.logical_and(p == 1, i == 0))
    def _():
        mean = acc_s[...] * (1.0 / N)
        var = acc_q[...] * (1.0 / N) - mean * mean
        g_rstd = gamma_ref[...] * lax.rsqrt(var + EPS)
        scale_s[...] = g_rstd
        shift_s[...] = beta_ref[...] - mean * g_rstd

    @pl.when(p == 1)
    def _():
        out_ref[...] = z_s[pl.ds(i * BM, BM), :] * scale_s[...] + shift_s[...]


_fin_call = pl.pallas_call(
    _fin_body,
    grid=(2, GRID),
    in_specs=[
        pl.BlockSpec((NC, BM, D), lambda p, i: (0, i, 0)),
        pl.BlockSpec((BM, D), lambda p, i: (i, 0)),
        pl.BlockSpec((NC, BM), lambda p, i: (0, i)),
        pl.BlockSpec((1, D), lambda p, i: (0, 0)),
        pl.BlockSpec((1, D), lambda p, i: (0, 0)),
        pl.BlockSpec((1, D), lambda p, i: (0, 0)),
    ],
    out_specs=pl.BlockSpec((BM, D), lambda p, i: (jnp.where(p == 0, 0, i), 0)),
    out_shape=jax.ShapeDtypeStruct((N, D), jnp.float32),
    scratch_shapes=[
        pltpu.VMEM((GRID * BM, D), jnp.float32),
        pltpu.VMEM((1, D), jnp.float32),
        pltpu.VMEM((1, D), jnp.float32),
        pltpu.VMEM((1, D), jnp.float32),
        pltpu.VMEM((1, D), jnp.float32),
    ],
)


def kernel(x, edge_index, W, b, gamma, beta):
    npad = ECH2D * CH - E
    ar = jnp.arange(npad, dtype=jnp.int32)
    srcp = jnp.concatenate([edge_index[0], (ar * 37) % N])
    dstp = jnp.concatenate([edge_index[1], N + ar % (NP - N)])
    dst2d = dstp.reshape(ECH2D, CH)

    degp = _deg_kernel(dst2d).reshape(NC, NP)
    hs = _hs_call(degp, x, W)
    y = _agg_kernel(srcp, dstp, hs).reshape(NC, NP, D)
    return _fin_call(y, hs, degp, b.reshape(1, D), gamma.reshape(1, D),
                     beta.reshape(1, D))


# speedup vs baseline: 3.1110x; 1.0401x over previous
import functools

import jax
import jax.numpy as jnp
from jax import lax
from jax.experimental import pallas as pl
from jax.experimental.pallas import tpu as pltpu
from jax.experimental.pallas import tpu_sc as plsc

N = 10000
E = 320000
D = 128
EPS = 1e-5
NEG_SLOPE = 0.01

NC, NS = 2, 16
NW = NC * NS
CH = 128
ECH = E // CH
NCH = 80
ECH2D = NW * NCH

BM = 2048
GRID = (N + BM - 1) // BM
NP = 10240

_mesh = plsc.VectorSubcoreMesh(
    core_axis_name="c", subcore_axis_name="s", num_cores=NC, num_subcores=NS)


@functools.partial(
    pl.kernel,
    out_type=jax.ShapeDtypeStruct((NC * NP,), jnp.float32),
    mesh=_mesh,
    scratch_types=[
        pltpu.VMEM_SHARED((NP,), jnp.float32),
        pltpu.VMEM((NCH, CH), jnp.int32),
        pltpu.VMEM((CH,), jnp.float32),
        pltpu.VMEM((NP // NS,), jnp.float32),
    ],
)
def _deg_kernel(dst2d_hbm, out_hbm, acc, didx, ones, zbuf):
    c = lax.axis_index("c")
    s = lax.axis_index("s")
    wid = s * NC + c

    one16 = jnp.full((16,), 1.0, dtype=jnp.float32)
    zero16 = jnp.zeros((16,), dtype=jnp.float32)

    @pl.loop(0, CH // 16)
    def _(i):
        ones[pl.ds(i * 16, 16)] = one16

    @pl.loop(0, NP // NS // 16)
    def _(i):
        zbuf[pl.ds(i * 16, 16)] = zero16
    pltpu.sync_copy(zbuf, acc.at[pl.ds(s * (NP // NS), NP // NS)])

    pltpu.sync_copy(dst2d_hbm.at[pl.ds(wid * NCH, NCH)], didx)

    plsc.subcore_barrier()

    @pl.loop(0, NCH)
    def _(k):
        pltpu.sync_copy(ones, acc.at[didx.at[k]], add=True)

    plsc.subcore_barrier()

    pltpu.sync_copy(acc.at[pl.ds(s * (NP // NS), NP // NS)], zbuf)
    pltpu.sync_copy(zbuf, out_hbm.at[pl.ds(c * NP + s * (NP // NS), NP // NS)])


def _dinv_col(deg_ref):
    deg = 1.0 + deg_ref[0:1, :] + deg_ref[1:2, :]
    return jnp.reshape(lax.rsqrt(deg), (BM, 1))


def _hs_body(deg_ref, x_ref, w_ref, hs_ref):
    h = jnp.dot(x_ref[...], w_ref[...], preferred_element_type=jnp.float32)
    hs_ref[...] = h * _dinv_col(deg_ref)


_hs_call = pl.pallas_call(
    _hs_body,
    grid=(GRID,),
    in_specs=[
        pl.BlockSpec((NC, BM), lambda i: (0, i)),
        pl.BlockSpec((BM, D), lambda i: (i, 0)),
        pl.BlockSpec((D, D), lambda i: (0, 0)),
    ],
    out_specs=pl.BlockSpec((BM, D), lambda i: (i, 0)),
    out_shape=jax.ShapeDtypeStruct((N, D), jnp.float32),
)


@functools.partial(
    pl.kernel,
    out_type=jax.ShapeDtypeStruct((NC * NP, D), jnp.float32),
    mesh=_mesh,
    scratch_types=[
        pltpu.VMEM_SHARED((NP, D), jnp.float32),
        pltpu.VMEM((4, 2, CH), jnp.int32),
        pltpu.VMEM((CH, D), jnp.float32),
        pltpu.VMEM((CH, D), jnp.float32),
        pltpu.SemaphoreType.DMA,
        pltpu.SemaphoreType.DMA,
        pltpu.SemaphoreType.DMA,
        pltpu.SemaphoreType.DMA,
        pltpu.SemaphoreType.DMA,
        pltpu.SemaphoreType.DMA,
        pltpu.SemaphoreType.DMA,
        pltpu.SemaphoreType.DMA,
    ],
)
def _agg_kernel(src_hbm, dst_hbm, hs_hbm, out_hbm, acc, idx,
                rows0, rows1, i0, i1, i2, i3, g0, g1, s0, s1):
    c = lax.axis_index("c")
    s = lax.axis_index("s")
    wid = s * NC + c
    rbufs = (rows0, rows1)
    isems = (i0, i1, i2, i3)
    gsems = (g0, g1)
    ssems = (s0, s1)

    zero16 = jnp.zeros((16,), dtype=jnp.float32)

    @pl.loop(0, CH)
    def _(r):
        @pl.loop(0, D // 16)
        def _(j):
            rows0[r, pl.ds(j * 16, 16)] = zero16

    rbase = s * (NP // NS)
    for k in range(5):
        pltpu.sync_copy(rows0, acc.at[pl.ds(rbase + k * CH, CH)])

    plsc.subcore_barrier()

    ebase = wid * NCH * CH

    def _idx_start(chunk, q):
        pltpu.async_copy(src_hbm.at[pl.ds(ebase + chunk * CH, CH)],
                         idx.at[q, 0], isems[q])
        pltpu.async_copy(dst_hbm.at[pl.ds(ebase + chunk * CH, CH)],
                         idx.at[q, 1], isems[q])

    def _idx_wait(chunk, q):
        pltpu.make_async_copy(src_hbm.at[pl.ds(ebase + chunk * CH, CH)],
                              idx.at[q, 0], isems[q]).wait()
        pltpu.make_async_copy(dst_hbm.at[pl.ds(ebase + chunk * CH, CH)],
                              idx.at[q, 1], isems[q]).wait()

    def _gather_start(chunk, q, b):
        pltpu.async_copy(hs_hbm.at[idx.at[q, 0]], rbufs[b], gsems[b])

    def _gather_wait(chunk, q, b):
        pltpu.make_async_copy(hs_hbm.at[idx.at[q, 0]],
                              rbufs[b], gsems[b]).wait()

    def _scatter_start(chunk, q, b):
        pltpu.async_copy(rbufs[b], acc.at[idx.at[q, 1]], ssems[b], add=True)

    def _scatter_wait(chunk, q, b):
        pltpu.make_async_copy(rbufs[b], acc.at[idx.at[q, 1]],
                              ssems[b]).wait()

    def _steady(i, q, b):
        qm1 = (q + 3) % 4
        qm2 = (q + 2) % 4
        _idx_wait(i, q)
        _scatter_wait(i - 2, qm2, b)
        _gather_start(i, q, b)
        _gather_wait(i - 1, qm1, 1 - b)
        _scatter_start(i - 1, qm1, 1 - b)

        @pl.when(i + 2 < NCH)
        def _():
            _idx_start(i + 2, qm2)

    _idx_start(0, 0)
    _idx_start(1, 1)

    _idx_wait(0, 0)
    _gather_start(0, 0, 0)
    _idx_start(2, 2)

    _idx_wait(1, 1)
    _gather_start(1, 1, 1)
    _gather_wait(0, 0, 0)
    _scatter_start(0, 0, 0)
    _idx_start(3, 3)

    _steady(2, 2, 0)
    _steady(3, 3, 1)

    @pl.loop(0, (NCH - 4) // 4)
    def _(k):
        for j in range(4):
            _steady(4 + 4 * k + j, j, j % 2)

    _gather_wait(NCH - 1, (NCH - 1) % 4, (NCH - 1) % 2)
    _scatter_start(NCH - 1, (NCH - 1) % 4, (NCH - 1) % 2)
    _scatter_wait(NCH - 2, (NCH - 2) % 4, (NCH - 2) % 2)
    _scatter_wait(NCH - 1, (NCH - 1) % 4, (NCH - 1) % 2)

    plsc.subcore_barrier()

    for k in range(5):
        pltpu.sync_copy(acc.at[pl.ds(rbase + k * CH, CH)], rows0)
        pltpu.sync_copy(rows0, out_hbm.at[pl.ds(c * NP + rbase + k * CH, CH)])


def _fin_body(y_ref, hs_ref, deg_ref, b_ref, gamma_ref, beta_ref, out_ref,
              z_s, acc_s, acc_q, scale_s, shift_s):
    p = pl.program_id(0)
    i = pl.program_id(1)

    @pl.when(p == 0)
    def _():
        t = ((y_ref[0] + y_ref[1] + hs_ref[...]) * _dinv_col(deg_ref)
             + b_ref[...])
        z = jnp.where(t >= 0, t, NEG_SLOPE * t)
        z_s[pl.ds(i * BM, BM), :] = z

        @pl.when(i == 0)
        def _():
            acc_s[...] = jnp.zeros_like(acc_s)
            acc_q[...] = jnp.zeros_like(acc_q)

        valid = (i * BM + lax.iota(jnp.int32, BM)[:, None]) < N
        zm = jnp.where(valid, z, 0.0)
        acc_s[...] += jnp.sum(zm, axis=0, keepdims=True)
        acc_q[...] += jnp.sum(zm * zm, axis=0, keepdims=True)

    @pl.when(jnp.logical_and(p == 1, i == 0))
    def _():
        mean = acc_s[...] * (1.0 / N)
        var = acc_q[...] * (1.0 / N) - mean * mean
        g_rstd = gamma_ref[...] * lax.rsqrt(var + EPS)
        scale_s[...] = g_rstd
        shift_s[...] = beta_ref[...] - mean * g_rstd

    @pl.when(p == 1)
    def _():
        out_ref[...] = z_s[pl.ds(i * BM, BM), :] * scale_s[...] + shift_s[...]


_fin_call = pl.pallas_call(
    _fin_body,
    grid=(2, GRID),
    in_specs=[
        pl.BlockSpec((NC, BM, D), lambda p, i: (0, i, 0)),
        pl.BlockSpec((BM, D), lambda p, i: (i, 0)),
        pl.BlockSpec((NC, BM), lambda p, i: (0, i)),
        pl.BlockSpec((1, D), lambda p, i: (0, 0)),
        pl.BlockSpec((1, D), lambda p, i: (0, 0)),
        pl.BlockSpec((1, D), lambda p, i: (0, 0)),
    ],
    out_specs=pl.BlockSpec((BM, D), lambda p, i: (jnp.where(p == 0, 0, i), 0)),
    out_shape=jax.ShapeDtypeStruct((N, D), jnp.float32),
    scratch_shapes=[
        pltpu.VMEM((GRID * BM, D), jnp.float32),
        pltpu.VMEM((1, D), jnp.float32),
        pltpu.VMEM((1, D), jnp.float32),
        pltpu.VMEM((1, D), jnp.float32),
        pltpu.VMEM((1, D), jnp.float32),
    ],
)


def kernel(x, edge_index, W, b, gamma, beta):
    npad = ECH2D * CH - E
    ar = jnp.arange(npad, dtype=jnp.int32)
    srcp = jnp.concatenate([edge_index[0], (ar * 37) % N])
    dstp = jnp.concatenate([edge_index[1], N + ar % (NP - N)])
    dst2d = dstp.reshape(ECH2D, CH)

    degp = _deg_kernel(dst2d).reshape(NC, NP)
    hs = _hs_call(degp, x, W)
    y = _agg_kernel(srcp, dstp, hs).reshape(NC, NP, D)
    return _fin_call(y, hs, degp, b.reshape(1, D), gamma.reshape(1, D),
                     beta.reshape(1, D))


# speedup vs baseline: 3.1531x; 1.0136x over previous
import functools

import jax
import jax.numpy as jnp
from jax import lax
from jax.experimental import pallas as pl
from jax.experimental.pallas import tpu as pltpu
from jax.experimental.pallas import tpu_sc as plsc

N = 10000
E = 320000
D = 128
EPS = 1e-5
NEG_SLOPE = 0.01

NC, NS = 2, 16
NW = NC * NS
CH = 128
ECH = E // CH
NCH = 80
ECH2D = NW * NCH

BM = 2560
GRID = (N + BM - 1) // BM
NP = 10240

_mesh = plsc.VectorSubcoreMesh(
    core_axis_name="c", subcore_axis_name="s", num_cores=NC, num_subcores=NS)


@functools.partial(
    pl.kernel,
    out_type=jax.ShapeDtypeStruct((NC * NP,), jnp.float32),
    mesh=_mesh,
    scratch_types=[
        pltpu.VMEM_SHARED((NP,), jnp.float32),
        pltpu.VMEM((NCH, CH), jnp.int32),
        pltpu.VMEM((CH,), jnp.float32),
        pltpu.VMEM((NP // NS,), jnp.float32),
    ],
)
def _deg_kernel(dst2d_hbm, out_hbm, acc, didx, ones, zbuf):
    c = lax.axis_index("c")
    s = lax.axis_index("s")
    wid = s * NC + c

    one16 = jnp.full((16,), 1.0, dtype=jnp.float32)
    zero16 = jnp.zeros((16,), dtype=jnp.float32)

    @pl.loop(0, CH // 16)
    def _(i):
        ones[pl.ds(i * 16, 16)] = one16

    @pl.loop(0, NP // NS // 16)
    def _(i):
        zbuf[pl.ds(i * 16, 16)] = zero16
    pltpu.sync_copy(zbuf, acc.at[pl.ds(s * (NP // NS), NP // NS)])

    pltpu.sync_copy(dst2d_hbm.at[pl.ds(wid * NCH, NCH)], didx)

    plsc.subcore_barrier()

    @pl.loop(0, NCH)
    def _(k):
        pltpu.sync_copy(ones, acc.at[didx.at[k]], add=True)

    plsc.subcore_barrier()

    pltpu.sync_copy(acc.at[pl.ds(s * (NP // NS), NP // NS)], zbuf)
    pltpu.sync_copy(zbuf, out_hbm.at[pl.ds(c * NP + s * (NP // NS), NP // NS)])


def _dinv_col(deg_ref):
    deg = 1.0 + deg_ref[0:1, :] + deg_ref[1:2, :]
    return jnp.reshape(lax.rsqrt(deg), (BM, 1))


def _hs_body(deg_ref, x_ref, w_ref, hs_ref):
    h = jnp.dot(x_ref[...], w_ref[...], preferred_element_type=jnp.float32)
    hs_ref[...] = h * _dinv_col(deg_ref)


_hs_call = pl.pallas_call(
    _hs_body,
    grid=(GRID,),
    in_specs=[
        pl.BlockSpec((NC, BM), lambda i: (0, i)),
        pl.BlockSpec((BM, D), lambda i: (i, 0)),
        pl.BlockSpec((D, D), lambda i: (0, 0)),
    ],
    out_specs=pl.BlockSpec((BM, D), lambda i: (i, 0)),
    out_shape=jax.ShapeDtypeStruct((N, D), jnp.float32),
)


@functools.partial(
    pl.kernel,
    out_type=jax.ShapeDtypeStruct((NC * NP, D), jnp.float32),
    mesh=_mesh,
    scratch_types=[
        pltpu.VMEM_SHARED((NP, D), jnp.float32),
        pltpu.VMEM((4, 2, CH), jnp.int32),
        pltpu.VMEM((CH, D), jnp.float32),
        pltpu.VMEM((CH, D), jnp.float32),
        pltpu.SemaphoreType.DMA,
        pltpu.SemaphoreType.DMA,
        pltpu.SemaphoreType.DMA,
        pltpu.SemaphoreType.DMA,
        pltpu.SemaphoreType.DMA,
        pltpu.SemaphoreType.DMA,
        pltpu.SemaphoreType.DMA,
        pltpu.SemaphoreType.DMA,
    ],
)
def _agg_kernel(src_hbm, dst_hbm, hs_hbm, out_hbm, acc, idx,
                rows0, rows1, i0, i1, i2, i3, g0, g1, s0, s1):
    c = lax.axis_index("c")
    s = lax.axis_index("s")
    wid = s * NC + c
    rbufs = (rows0, rows1)
    isems = (i0, i1, i2, i3)
    gsems = (g0, g1)
    ssems = (s0, s1)

    zero16 = jnp.zeros((16,), dtype=jnp.float32)

    @pl.loop(0, CH)
    def _(r):
        @pl.loop(0, D // 16)
        def _(j):
            rows0[r, pl.ds(j * 16, 16)] = zero16

    rbase = s * (NP // NS)
    for k in range(5):
        pltpu.sync_copy(rows0, acc.at[pl.ds(rbase + k * CH, CH)])

    plsc.subcore_barrier()

    ebase = wid * NCH * CH

    def _idx_start(chunk, q):
        pltpu.async_copy(src_hbm.at[pl.ds(ebase + chunk * CH, CH)],
                         idx.at[q, 0], isems[q])
        pltpu.async_copy(dst_hbm.at[pl.ds(ebase + chunk * CH, CH)],
                         idx.at[q, 1], isems[q])

    def _idx_wait(chunk, q):
        pltpu.make_async_copy(src_hbm.at[pl.ds(ebase + chunk * CH, CH)],
                              idx.at[q, 0], isems[q]).wait()
        pltpu.make_async_copy(dst_hbm.at[pl.ds(ebase + chunk * CH, CH)],
                              idx.at[q, 1], isems[q]).wait()

    def _gather_start(chunk, q, b):
        pltpu.async_copy(hs_hbm.at[idx.at[q, 0]], rbufs[b], gsems[b])

    def _gather_wait(chunk, q, b):
        pltpu.make_async_copy(hs_hbm.at[idx.at[q, 0]],
                              rbufs[b], gsems[b]).wait()

    def _scatter_start(chunk, q, b):
        pltpu.async_copy(rbufs[b], acc.at[idx.at[q, 1]], ssems[b], add=True)

    def _scatter_wait(chunk, q, b):
        pltpu.make_async_copy(rbufs[b], acc.at[idx.at[q, 1]],
                              ssems[b]).wait()

    def _steady(i, q, b):
        qm1 = (q + 3) % 4
        qm2 = (q + 2) % 4
        _idx_wait(i, q)
        _scatter_wait(i - 2, qm2, b)
        _gather_start(i, q, b)
        _gather_wait(i - 1, qm1, 1 - b)
        _scatter_start(i - 1, qm1, 1 - b)

        @pl.when(i + 2 < NCH)
        def _():
            _idx_start(i + 2, qm2)

    _idx_start(0, 0)
    _idx_start(1, 1)

    _idx_wait(0, 0)
    _gather_start(0, 0, 0)
    _idx_start(2, 2)

    _idx_wait(1, 1)
    _gather_start(1, 1, 1)
    _gather_wait(0, 0, 0)
    _scatter_start(0, 0, 0)
    _idx_start(3, 3)

    _steady(2, 2, 0)
    _steady(3, 3, 1)

    @pl.loop(0, (NCH - 4) // 4)
    def _(k):
        for j in range(4):
            _steady(4 + 4 * k + j, j, j % 2)

    _gather_wait(NCH - 1, (NCH - 1) % 4, (NCH - 1) % 2)
    _scatter_start(NCH - 1, (NCH - 1) % 4, (NCH - 1) % 2)
    _scatter_wait(NCH - 2, (NCH - 2) % 4, (NCH - 2) % 2)
    _scatter_wait(NCH - 1, (NCH - 1) % 4, (NCH - 1) % 2)

    plsc.subcore_barrier()

    for k in range(5):
        pltpu.sync_copy(acc.at[pl.ds(rbase + k * CH, CH)], rows0)
        pltpu.sync_copy(rows0, out_hbm.at[pl.ds(c * NP + rbase + k * CH, CH)])


def _fin_body(y_ref, hs_ref, deg_ref, b_ref, gamma_ref, beta_ref, out_ref,
              z_s, acc_s, acc_q, scale_s, shift_s):
    p = pl.program_id(0)
    i = pl.program_id(1)

    @pl.when(p == 0)
    def _():
        t = ((y_ref[0] + y_ref[1] + hs_ref[...]) * _dinv_col(deg_ref)
             + b_ref[...])
        z = jnp.where(t >= 0, t, NEG_SLOPE * t)
        z_s[pl.ds(i * BM, BM), :] = z

        @pl.when(i == 0)
        def _():
            acc_s[...] = jnp.zeros_like(acc_s)
            acc_q[...] = jnp.zeros_like(acc_q)

        valid = (i * BM + lax.iota(jnp.int32, BM)[:, None]) < N
        zm = jnp.where(valid, z, 0.0)
        acc_s[...] += jnp.sum(zm, axis=0, keepdims=True)
        acc_q[...] += jnp.sum(zm * zm, axis=0, keepdims=True)

    @pl.when(jnp.logical_and(p == 1, i == 0))
    def _():
        mean = acc_s[...] * (1.0 / N)
        var = acc_q[...] * (1.0 / N) - mean * mean
        g_rstd = gamma_ref[...] * lax.rsqrt(var + EPS)
        scale_s[...] = g_rstd
        shift_s[...] = beta_ref[...] - mean * g_rstd

    @pl.when(p == 1)
    def _():
        out_ref[...] = z_s[pl.ds(i * BM, BM), :] * scale_s[...] + shift_s[...]


_fin_call = pl.pallas_call(
    _fin_body,
    grid=(2, GRID),
    in_specs=[
        pl.BlockSpec((NC, BM, D), lambda p, i: (0, i, 0)),
        pl.BlockSpec((BM, D), lambda p, i: (i, 0)),
        pl.BlockSpec((NC, BM), lambda p, i: (0, i)),
        pl.BlockSpec((1, D), lambda p, i: (0, 0)),
        pl.BlockSpec((1, D), lambda p, i: (0, 0)),
        pl.BlockSpec((1, D), lambda p, i: (0, 0)),
    ],
    out_specs=pl.BlockSpec((BM, D), lambda p, i: (jnp.where(p == 0, 0, i), 0)),
    out_shape=jax.ShapeDtypeStruct((N, D), jnp.float32),
    scratch_shapes=[
        pltpu.VMEM((GRID * BM, D), jnp.float32),
        pltpu.VMEM((1, D), jnp.float32),
        pltpu.VMEM((1, D), jnp.float32),
        pltpu.VMEM((1, D), jnp.float32),
        pltpu.VMEM((1, D), jnp.float32),
    ],
)


def kernel(x, edge_index, W, b, gamma, beta):
    npad = ECH2D * CH - E
    ar = jnp.arange(npad, dtype=jnp.int32)
    srcp = jnp.concatenate([edge_index[0], (ar * 37) % N])
    dstp = jnp.concatenate([edge_index[1], N + ar % (NP - N)])
    dst2d = dstp.reshape(ECH2D, CH)

    degp = _deg_kernel(dst2d).reshape(NC, NP)
    hs = _hs_call(degp, x, W)
    y = _agg_kernel(srcp, dstp, hs).reshape(NC, NP, D)
    return _fin_call(y, hs, degp, b.reshape(1, D), gamma.reshape(1, D),
                     beta.reshape(1, D))


# speedup vs baseline: 3.2115x; 1.0185x over previous
import functools

import jax
import jax.numpy as jnp
from jax import lax
from jax.experimental import pallas as pl
from jax.experimental.pallas import tpu as pltpu
from jax.experimental.pallas import tpu_sc as plsc

N = 10000
E = 320000
D = 128
EPS = 1e-5
NEG_SLOPE = 0.01

NC, NS = 2, 16
NW = NC * NS
CH = 128
ECH = E // CH
NCH = 80
ECH2D = NW * NCH

BM = 5120
GRID = (N + BM - 1) // BM
NP = 10240

_mesh = plsc.VectorSubcoreMesh(
    core_axis_name="c", subcore_axis_name="s", num_cores=NC, num_subcores=NS)


@functools.partial(
    pl.kernel,
    out_type=jax.ShapeDtypeStruct((NC * NP,), jnp.float32),
    mesh=_mesh,
    scratch_types=[
        pltpu.VMEM_SHARED((NP,), jnp.float32),
        pltpu.VMEM((NCH, CH), jnp.int32),
        pltpu.VMEM((CH,), jnp.float32),
        pltpu.VMEM((NP // NS,), jnp.float32),
    ],
)
def _deg_kernel(dst2d_hbm, out_hbm, acc, didx, ones, zbuf):
    c = lax.axis_index("c")
    s = lax.axis_index("s")
    wid = s * NC + c

    one16 = jnp.full((16,), 1.0, dtype=jnp.float32)
    zero16 = jnp.zeros((16,), dtype=jnp.float32)

    @pl.loop(0, CH // 16)
    def _(i):
        ones[pl.ds(i * 16, 16)] = one16

    @pl.loop(0, NP // NS // 16)
    def _(i):
        zbuf[pl.ds(i * 16, 16)] = zero16
    pltpu.sync_copy(zbuf, acc.at[pl.ds(s * (NP // NS), NP // NS)])

    pltpu.sync_copy(dst2d_hbm.at[pl.ds(wid * NCH, NCH)], didx)

    plsc.subcore_barrier()

    @pl.loop(0, NCH)
    def _(k):
        pltpu.sync_copy(ones, acc.at[didx.at[k]], add=True)

    plsc.subcore_barrier()

    pltpu.sync_copy(acc.at[pl.ds(s * (NP // NS), NP // NS)], zbuf)
    pltpu.sync_copy(zbuf, out_hbm.at[pl.ds(c * NP + s * (NP // NS), NP // NS)])


def _dinv_col(deg_ref):
    deg = 1.0 + deg_ref[0:1, :] + deg_ref[1:2, :]
    return jnp.reshape(lax.rsqrt(deg), (BM, 1))


def _hs_body(deg_ref, x_ref, w_ref, hs_ref):
    h = jnp.dot(x_ref[...], w_ref[...], preferred_element_type=jnp.float32)
    hs_ref[...] = h * _dinv_col(deg_ref)


_hs_call = pl.pallas_call(
    _hs_body,
    grid=(GRID,),
    in_specs=[
        pl.BlockSpec((NC, BM), lambda i: (0, i)),
        pl.BlockSpec((BM, D), lambda i: (i, 0)),
        pl.BlockSpec((D, D), lambda i: (0, 0)),
    ],
    out_specs=pl.BlockSpec((BM, D), lambda i: (i, 0)),
    out_shape=jax.ShapeDtypeStruct((N, D), jnp.float32),
)


@functools.partial(
    pl.kernel,
    out_type=jax.ShapeDtypeStruct((NC * NP, D), jnp.float32),
    mesh=_mesh,
    scratch_types=[
        pltpu.VMEM_SHARED((NP, D), jnp.float32),
        pltpu.VMEM((4, 2, CH), jnp.int32),
        pltpu.VMEM((CH, D), jnp.float32),
        pltpu.VMEM((CH, D), jnp.float32),
        pltpu.SemaphoreType.DMA,
        pltpu.SemaphoreType.DMA,
        pltpu.SemaphoreType.DMA,
        pltpu.SemaphoreType.DMA,
        pltpu.SemaphoreType.DMA,
        pltpu.SemaphoreType.DMA,
        pltpu.SemaphoreType.DMA,
        pltpu.SemaphoreType.DMA,
    ],
)
def _agg_kernel(src_hbm, dst_hbm, hs_hbm, out_hbm, acc, idx,
                rows0, rows1, i0, i1, i2, i3, g0, g1, s0, s1):
    c = lax.axis_index("c")
    s = lax.axis_index("s")
    wid = s * NC + c
    rbufs = (rows0, rows1)
    isems = (i0, i1, i2, i3)
    gsems = (g0, g1)
    ssems = (s0, s1)

    zero16 = jnp.zeros((16,), dtype=jnp.float32)

    @pl.loop(0, CH)
    def _(r):
        @pl.loop(0, D // 16)
        def _(j):
            rows0[r, pl.ds(j * 16, 16)] = zero16

    rbase = s * (NP // NS)
    for k in range(5):
        pltpu.sync_copy(rows0, acc.at[pl.ds(rbase + k * CH, CH)])

    plsc.subcore_barrier()

    ebase = wid * NCH * CH

    def _idx_start(chunk, q):
        pltpu.async_copy(src_hbm.at[pl.ds(ebase + chunk * CH, CH)],
                         idx.at[q, 0], isems[q])
        pltpu.async_copy(dst_hbm.at[pl.ds(ebase + chunk * CH, CH)],
                         idx.at[q, 1], isems[q])

    def _idx_wait(chunk, q):
        pltpu.make_async_copy(src_hbm.at[pl.ds(ebase + chunk * CH, CH)],
                              idx.at[q, 0], isems[q]).wait()
        pltpu.make_async_copy(dst_hbm.at[pl.ds(ebase + chunk * CH, CH)],
                              idx.at[q, 1], isems[q]).wait()

    def _gather_start(chunk, q, b):
        pltpu.async_copy(hs_hbm.at[idx.at[q, 0]], rbufs[b], gsems[b])

    def _gather_wait(chunk, q, b):
        pltpu.make_async_copy(hs_hbm.at[idx.at[q, 0]],
                              rbufs[b], gsems[b]).wait()

    def _scatter_start(chunk, q, b):
        pltpu.async_copy(rbufs[b], acc.at[idx.at[q, 1]], ssems[b], add=True)

    def _scatter_wait(chunk, q, b):
        pltpu.make_async_copy(rbufs[b], acc.at[idx.at[q, 1]],
                              ssems[b]).wait()

    def _steady(i, q, b):
        qm1 = (q + 3) % 4
        qm2 = (q + 2) % 4
        _idx_wait(i, q)
        _scatter_wait(i - 2, qm2, b)
        _gather_start(i, q, b)
        _gather_wait(i - 1, qm1, 1 - b)
        _scatter_start(i - 1, qm1, 1 - b)

        @pl.when(i + 2 < NCH)
        def _():
            _idx_start(i + 2, qm2)

    _idx_start(0, 0)
    _idx_start(1, 1)

    _idx_wait(0, 0)
    _gather_start(0, 0, 0)
    _idx_start(2, 2)

    _idx_wait(1, 1)
    _gather_start(1, 1, 1)
    _gather_wait(0, 0, 0)
    _scatter_start(0, 0, 0)
    _idx_start(3, 3)

    _steady(2, 2, 0)
    _steady(3, 3, 1)

    @pl.loop(0, (NCH - 4) // 4)
    def _(k):
        for j in range(4):
            _steady(4 + 4 * k + j, j, j % 2)

    _gather_wait(NCH - 1, (NCH - 1) % 4, (NCH - 1) % 2)
    _scatter_start(NCH - 1, (NCH - 1) % 4, (NCH - 1) % 2)
    _scatter_wait(NCH - 2, (NCH - 2) % 4, (NCH - 2) % 2)
    _scatter_wait(NCH - 1, (NCH - 1) % 4, (NCH - 1) % 2)

    plsc.subcore_barrier()

    for k in range(5):
        pltpu.sync_copy(acc.at[pl.ds(rbase + k * CH, CH)], rows0)
        pltpu.sync_copy(rows0, out_hbm.at[pl.ds(c * NP + rbase + k * CH, CH)])


def _fin_body(y_ref, hs_ref, deg_ref, b_ref, gamma_ref, beta_ref, out_ref,
              z_s, acc_s, acc_q, scale_s, shift_s):
    p = pl.program_id(0)
    i = pl.program_id(1)

    @pl.when(p == 0)
    def _():
        t = ((y_ref[0] + y_ref[1] + hs_ref[...]) * _dinv_col(deg_ref)
             + b_ref[...])
        z = jnp.where(t >= 0, t, NEG_SLOPE * t)
        z_s[pl.ds(i * BM, BM), :] = z

        @pl.when(i == 0)
        def _():
            acc_s[...] = jnp.zeros_like(acc_s)
            acc_q[...] = jnp.zeros_like(acc_q)

        valid = (i * BM + lax.iota(jnp.int32, BM)[:, None]) < N
        zm = jnp.where(valid, z, 0.0)
        acc_s[...] += jnp.sum(zm, axis=0, keepdims=True)
        acc_q[...] += jnp.sum(zm * zm, axis=0, keepdims=True)

    @pl.when(jnp.logical_and(p == 1, i == 0))
    def _():
        mean = acc_s[...] * (1.0 / N)
        var = acc_q[...] * (1.0 / N) - mean * mean
        g_rstd = gamma_ref[...] * lax.rsqrt(var + EPS)
        scale_s[...] = g_rstd
        shift_s[...] = beta_ref[...] - mean * g_rstd

    @pl.when(p == 1)
    def _():
        out_ref[...] = z_s[pl.ds(i * BM, BM), :] * scale_s[...] + shift_s[...]


_fin_call = pl.pallas_call(
    _fin_body,
    grid=(2, GRID),
    in_specs=[
        pl.BlockSpec((NC, BM, D), lambda p, i: (0, i, 0)),
        pl.BlockSpec((BM, D), lambda p, i: (i, 0)),
        pl.BlockSpec((NC, BM), lambda p, i: (0, i)),
        pl.BlockSpec((1, D), lambda p, i: (0, 0)),
        pl.BlockSpec((1, D), lambda p, i: (0, 0)),
        pl.BlockSpec((1, D), lambda p, i: (0, 0)),
    ],
    out_specs=pl.BlockSpec((BM, D), lambda p, i: (jnp.where(p == 0, 0, i), 0)),
    out_shape=jax.ShapeDtypeStruct((N, D), jnp.float32),
    scratch_shapes=[
        pltpu.VMEM((GRID * BM, D), jnp.float32),
        pltpu.VMEM((1, D), jnp.float32),
        pltpu.VMEM((1, D), jnp.float32),
        pltpu.VMEM((1, D), jnp.float32),
        pltpu.VMEM((1, D), jnp.float32),
    ],
)


def kernel(x, edge_index, W, b, gamma, beta):
    npad = ECH2D * CH - E
    ar = jnp.arange(npad, dtype=jnp.int32)
    srcp = jnp.concatenate([edge_index[0], (ar * 37) % N])
    dstp = jnp.concatenate([edge_index[1], N + ar % (NP - N)])
    dst2d = dstp.reshape(ECH2D, CH)

    degp = _deg_kernel(dst2d).reshape(NC, NP)
    hs = _hs_call(degp, x, W)
    y = _agg_kernel(srcp, dstp, hs).reshape(NC, NP, D)
    return _fin_call(y, hs, degp, b.reshape(1, D), gamma.reshape(1, D),
                     beta.reshape(1, D))
